# Initial kernel scaffold; baseline (speedup 1.0000x reference)
import functools

import jax
import jax.numpy as jnp
from jax import lax
from jax.experimental import pallas as pl
from jax.experimental.pallas import tpu as pltpu
from jax.experimental.pallas import tpu_sc as plsc

N = 10000
E = 320000
D = 128
H = 256
HH = H // 2

NTILE = 16
ROWS_PER_TILE = N // NTILE
EDGES_PER_TILE = E // NTILE
CH = 80
NWIN = EDGES_PER_TILE // CH

DEG_PAD = 10240
DEG_PER_TILE = DEG_PAD // NTILE
DEG_EDGES_PER_TILE = E // (2 * NTILE)
DEG_NWIN = DEG_EDGES_PER_TILE // CH

R = 400
GB = N // R

_sc_mesh = plsc.VectorSubcoreMesh(core_axis_name="c", subcore_axis_name="s")


@functools.partial(
    pl.kernel,
    out_type=jax.ShapeDtypeStruct((2 * DEG_PAD,), jnp.float32),
    mesh=_sc_mesh,
    scratch_types=[
        pltpu.VMEM_SHARED((DEG_PAD,), jnp.float32),
        pltpu.VMEM((CH,), jnp.int32),
        pltpu.VMEM((CH,), jnp.float32),
        pltpu.VMEM((DEG_PER_TILE,), jnp.float32),
    ],
)
def _deg_sc(dst_ref, part_ref, acc, idxb, ones_b, zbuf):
    c = lax.axis_index("c")
    sid = lax.axis_index("s")

    def fill_z(i, _):
        zbuf[pl.ds(i * 16, 16)] = jnp.zeros((16,), jnp.float32)
        return 0

    lax.fori_loop(0, DEG_PER_TILE // 16, fill_z, 0)

    def fill_o(i, _):
        ones_b[pl.ds(i * 16, 16)] = jnp.ones((16,), jnp.float32)
        return 0

    lax.fori_loop(0, CH // 16, fill_o, 0)

    my0 = pl.multiple_of(sid * DEG_PER_TILE, 8)
    pltpu.sync_copy(zbuf, acc.at[pl.ds(my0, DEG_PER_TILE)])
    plsc.subcore_barrier()

    base = c * (E // 2) + sid * DEG_EDGES_PER_TILE

    def win(w, _):
        off = pl.multiple_of(base + w * CH, 8)
        pltpu.sync_copy(dst_ref.at[pl.ds(off, CH)], idxb)
        pltpu.sync_copy(ones_b, acc.at[idxb], add=True)
        return 0

    lax.fori_loop(0, DEG_NWIN, win, 0)
    plsc.subcore_barrier()
    out0 = pl.multiple_of(c * DEG_PAD + sid * DEG_PER_TILE, 8)
    pltpu.sync_copy(acc.at[pl.ds(my0, DEG_PER_TILE)],
                    part_ref.at[pl.ds(out0, DEG_PER_TILE)])


@functools.partial(
    pl.kernel,
    out_type=jax.ShapeDtypeStruct((2 * N, HH), jnp.float32),
    mesh=_sc_mesh,
    scratch_types=[
        pltpu.VMEM_SHARED((N, HH), jnp.float32),
        pltpu.VMEM((CH,), jnp.int32),
        pltpu.VMEM((CH,), jnp.int32),
        pltpu.VMEM((CH, HH), jnp.float32),
        pltpu.SemaphoreType.DMA,
    ],
)
def _conv_sc(gcat_ref, src2_ref, dst2_ref, scat_ref, acc, idxs, idxd, buf, sem):
    c = lax.axis_index("c")
    sid = lax.axis_index("s")

    r0 = sid * ROWS_PER_TILE
    grow0 = c * N + sid * ROWS_PER_TILE
    pltpu.sync_copy(gcat_ref.at[pl.ds(grow0, ROWS_PER_TILE)],
                    acc.at[pl.ds(r0, ROWS_PER_TILE)])
    plsc.subcore_barrier()

    ebase = c * E + sid * EDGES_PER_TILE

    def win(w, _):
        off = pl.multiple_of(ebase + w * CH, 8)
        pltpu.sync_copy(src2_ref.at[pl.ds(off, CH)], idxs)
        pltpu.sync_copy(dst2_ref.at[pl.ds(off, CH)], idxd)
        pltpu.async_copy(gcat_ref.at[idxs], buf, sem).wait()
        pltpu.sync_copy(buf, acc.at[idxd], add=True)
        return 0

    lax.fori_loop(0, NWIN, win, 0)
    plsc.subcore_barrier()
    pltpu.sync_copy(acc.at[pl.ds(r0, ROWS_PER_TILE)],
                    scat_ref.at[pl.ds(grow0, ROWS_PER_TILE)])


def _t1_body(x_ref, d0_ref, d1_ref, nu_ref, wga_ref, bga_ref, wgb_ref,
             bgb_ref, w1_ref, b1_ref, g_ref, dis_ref):
    nu = nu_ref[0, 0]
    t = jnp.maximum(nu * wga_ref[...] + bga_ref[...], 0.0)
    logits = jnp.dot(t, wgb_ref[...], preferred_element_type=jnp.float32)
    logits = logits + bgb_ref[...]
    m = jnp.max(logits, axis=-1, keepdims=True)
    ex = jnp.exp(logits - m)
    fw = ex / jnp.sum(ex, axis=-1, keepdims=True)
    h0 = x_ref[...] * fw
    u = jnp.dot(h0, w1_ref[...], preferred_element_type=jnp.float32)
    u = u + b1_ref[...]
    deg = d0_ref[...] + d1_ref[...] + 1.0
    dis = lax.rsqrt(jnp.maximum(deg, 1.0))
    dis_ref[...] = dis
    g = u * dis
    g_ref[0] = g[:, :HH]
    g_ref[1] = g[:, HH:]


def _t23_body(s_ref, dis_ref, w_ref, b_ref, g_ref):
    dis = dis_ref[...]
    s_full = jnp.concatenate([s_ref[0], s_ref[1]], axis=1)
    h = jnp.maximum(s_full * dis, 0.0)
    u = jnp.dot(h, w_ref[...], preferred_element_type=jnp.float32)
    u = u + b_ref[...]
    g = u * dis
    g_ref[0] = g[:, :HH]
    g_ref[1] = g[:, HH:]


def _t4_body(s_ref, dis_ref, nu_ref, wa1h_ref, wa1nu_ref, ba1_ref, wa2_ref,
             ba2_ref, wo1_ref, bo1_ref, wo2_ref, bo2_ref, out_ref):
    dis = dis_ref[...]
    h3 = jnp.concatenate([s_ref[0], s_ref[1]], axis=1) * dis
    nu = nu_ref[0, 0]
    a = jnp.dot(h3, wa1h_ref[...], preferred_element_type=jnp.float32)
    a = jnp.maximum(a + nu * wa1nu_ref[...] + ba1_ref[...], 0.0)
    att_l = jnp.dot(a, wa2_ref[...], preferred_element_type=jnp.float32)
    att = jax.nn.sigmoid(att_l + ba2_ref[...])
    ah = h3 * att
    z = jnp.dot(ah, wo1_ref[...], preferred_element_type=jnp.float32)
    z = jnp.maximum(z + bo1_ref[...], 0.0)
    o = jnp.dot(z, wo2_ref[...], preferred_element_type=jnp.float32)
    out_ref[...] = jax.nn.sigmoid(o + bo2_ref[...])


def _full(shape):
    return pl.BlockSpec(shape, lambda i: tuple(0 for _ in shape))


_t1 = pl.pallas_call(
    _t1_body,
    grid=(GB,),
    in_specs=[
        pl.BlockSpec((R, D), lambda i: (i, 0)),
        pl.BlockSpec((R, 1), lambda i: (i, 0)),
        pl.BlockSpec((R, 1), lambda i: (i, 0)),
        _full((1, 1)),
        _full((1, 32)),
        _full((1, 32)),
        _full((32, D)),
        _full((1, D)),
        _full((D, H)),
        _full((1, H)),
    ],
    out_specs=[
        pl.BlockSpec((2, R, HH), lambda i: (0, i, 0)),
        pl.BlockSpec((R, 1), lambda i: (i, 0)),
    ],
    out_shape=[
        jax.ShapeDtypeStruct((2, N, HH), jnp.float32),
        jax.ShapeDtypeStruct((N, 1), jnp.float32),
    ],
)

_t23 = pl.pallas_call(
    _t23_body,
    grid=(GB,),
    in_specs=[
        pl.BlockSpec((2, R, HH), lambda i: (0, i, 0)),
        pl.BlockSpec((R, 1), lambda i: (i, 0)),
        _full((H, H)),
        _full((1, H)),
    ],
    out_specs=pl.BlockSpec((2, R, HH), lambda i: (0, i, 0)),
    out_shape=jax.ShapeDtypeStruct((2, N, HH), jnp.float32),
)

_t4 = pl.pallas_call(
    _t4_body,
    grid=(GB,),
    in_specs=[
        pl.BlockSpec((2, R, HH), lambda i: (0, i, 0)),
        pl.BlockSpec((R, 1), lambda i: (i, 0)),
        _full((1, 1)),
        _full((H, HH)),
        _full((1, HH)),
        _full((1, HH)),
        _full((HH, 1)),
        _full((1, 1)),
        _full((H, HH)),
        _full((1, HH)),
        _full((HH, 1)),
        _full((1, 1)),
    ],
    out_specs=pl.BlockSpec((R, 1), lambda i: (i, 0)),
    out_shape=jax.ShapeDtypeStruct((N, 1), jnp.float32),
)


def kernel(x, edge_index, nu, Wg_a, bg_a, Wg_b, bg_b, W1, b1, W2, b2, W3, b3,
           Wa1, ba1, Wa2, ba2, Wo1, bo1, Wo2, bo2):
    src = edge_index[0].astype(jnp.int32)
    dst = edge_index[1].astype(jnp.int32)
    src2 = jnp.concatenate([src, src + N])
    dst2 = jnp.concatenate([dst, dst])

    part = _deg_sc(dst)
    d0 = part[0:N].reshape(N, 1)
    d1 = part[DEG_PAD:DEG_PAD + N].reshape(N, 1)

    nu2 = nu.reshape(1, 1)
    g, dis = _t1(x, d0, d1, nu2, Wg_a, bg_a.reshape(1, 32), Wg_b,
                 bg_b.reshape(1, D), W1, b1.reshape(1, H))

    s = _conv_sc(g.reshape(2 * N, HH), src2, dst2).reshape(2, N, HH)
    g = _t23(s, dis, W2, b2.reshape(1, H))
    s = _conv_sc(g.reshape(2 * N, HH), src2, dst2).reshape(2, N, HH)
    g = _t23(s, dis, W3, b3.reshape(1, H))
    s = _conv_sc(g.reshape(2 * N, HH), src2, dst2).reshape(2, N, HH)

    out = _t4(s, dis, nu2, Wa1[:H], Wa1[H:H + 1], ba1.reshape(1, HH),
              Wa2, ba2.reshape(1, 1), Wo1, bo1.reshape(1, HH),
              Wo2, bo2.reshape(1, 1))
    return out


# speedup vs baseline: 7.2988x; 7.2988x over previous
import functools

import jax
import jax.numpy as jnp
from jax import lax
from jax.experimental import pallas as pl
from jax.experimental.pallas import tpu as pltpu
from jax.experimental.pallas import tpu_sc as plsc

N = 10000
NP = 10240
E = 320000
D = 128
H = 256
HH = H // 2

NTILE = 16
ROWS_PER_TILE = NP // NTILE
EDGES_PER_TILE = E // NTILE
CH = 80
NWIN = EDGES_PER_TILE // CH

DEG_PER_TILE = NP // NTILE
DEG_EDGES_PER_TILE = E // (2 * NTILE)
DEG_NWIN = DEG_EDGES_PER_TILE // CH

R = 320
GB = NP // R

_sc_mesh = plsc.VectorSubcoreMesh(core_axis_name="c", subcore_axis_name="s")


@functools.partial(
    pl.kernel,
    out_type=jax.ShapeDtypeStruct((2 * NP,), jnp.float32),
    mesh=_sc_mesh,
    scratch_types=[
        pltpu.VMEM_SHARED((NP,), jnp.float32),
        pltpu.VMEM((CH,), jnp.int32),
        pltpu.VMEM((CH,), jnp.float32),
        pltpu.VMEM((DEG_PER_TILE,), jnp.float32),
    ],
)
def _deg_sc(dst_ref, part_ref, acc, idxb, ones_b, zbuf):
    c = lax.axis_index("c")
    sid = lax.axis_index("s")

    def fill_z(i, _):
        zbuf[pl.ds(i * 16, 16)] = jnp.zeros((16,), jnp.float32)
        return 0

    lax.fori_loop(0, DEG_PER_TILE // 16, fill_z, 0)

    def fill_o(i, _):
        ones_b[pl.ds(i * 16, 16)] = jnp.ones((16,), jnp.float32)
        return 0

    lax.fori_loop(0, CH // 16, fill_o, 0)

    my0 = pl.multiple_of(sid * DEG_PER_TILE, 8)
    pltpu.sync_copy(zbuf, acc.at[pl.ds(my0, DEG_PER_TILE)])
    plsc.subcore_barrier()

    base = c * (E // 2) + sid * DEG_EDGES_PER_TILE

    def win(w, _):
        off = pl.multiple_of(base + w * CH, 8)
        pltpu.sync_copy(dst_ref.at[pl.ds(off, CH)], idxb)
        pltpu.sync_copy(ones_b, acc.at[idxb], add=True)
        return 0

    lax.fori_loop(0, DEG_NWIN, win, 0)
    plsc.subcore_barrier()
    out0 = pl.multiple_of(c * NP + sid * DEG_PER_TILE, 8)
    pltpu.sync_copy(acc.at[pl.ds(my0, DEG_PER_TILE)],
                    part_ref.at[pl.ds(out0, DEG_PER_TILE)])


@functools.partial(
    pl.kernel,
    out_type=jax.ShapeDtypeStruct((2 * NP, HH), jnp.float32),
    mesh=_sc_mesh,
    scratch_types=[
        pltpu.VMEM_SHARED((NP, HH), jnp.float32),
        pltpu.VMEM((CH,), jnp.int32),
        pltpu.VMEM((CH,), jnp.int32),
        pltpu.VMEM((CH, HH), jnp.float32),
        pltpu.SemaphoreType.DMA,
    ],
)
def _conv_sc(gcat_ref, src2_ref, dst2_ref, scat_ref, acc, idxs, idxd, buf, sem):
    c = lax.axis_index("c")
    sid = lax.axis_index("s")

    r0 = pl.multiple_of(sid * ROWS_PER_TILE, 8)
    grow0 = pl.multiple_of(c * NP + sid * ROWS_PER_TILE, 8)
    pltpu.sync_copy(gcat_ref.at[pl.ds(grow0, ROWS_PER_TILE)],
                    acc.at[pl.ds(r0, ROWS_PER_TILE)])
    plsc.subcore_barrier()

    ebase = c * E + sid * EDGES_PER_TILE

    def win(w, _):
        off = pl.multiple_of(ebase + w * CH, 8)
        pltpu.sync_copy(src2_ref.at[pl.ds(off, CH)], idxs)
        pltpu.sync_copy(dst2_ref.at[pl.ds(off, CH)], idxd)
        pltpu.async_copy(gcat_ref.at[idxs], buf, sem).wait()
        pltpu.sync_copy(buf, acc.at[idxd], add=True)
        return 0

    lax.fori_loop(0, NWIN, win, 0)
    plsc.subcore_barrier()
    pltpu.sync_copy(acc.at[pl.ds(r0, ROWS_PER_TILE)],
                    scat_ref.at[pl.ds(grow0, ROWS_PER_TILE)])


def _t1_body(x_ref, d0_ref, d1_ref, nu_ref, wga_ref, bga_ref, wgb_ref,
             bgb_ref, w1_ref, b1_ref, g_ref, dis_ref):
    nu = nu_ref[0, 0]
    t = jnp.maximum(nu * wga_ref[...] + bga_ref[...], 0.0)
    logits = jnp.dot(t, wgb_ref[...], preferred_element_type=jnp.float32)
    logits = logits + bgb_ref[...]
    m = jnp.max(logits, axis=-1, keepdims=True)
    ex = jnp.exp(logits - m)
    fw = ex / jnp.sum(ex, axis=-1, keepdims=True)
    h0 = x_ref[...] * fw
    u = jnp.dot(h0, w1_ref[...], preferred_element_type=jnp.float32)
    u = u + b1_ref[...]
    deg = d0_ref[...] + d1_ref[...] + 1.0
    dis = lax.rsqrt(jnp.maximum(deg, 1.0))
    dis_ref[...] = dis
    g = u * dis
    g_ref[0] = g[:, :HH]
    g_ref[1] = g[:, HH:]


def _t23_body(s_ref, dis_ref, w_ref, b_ref, g_ref):
    dis = dis_ref[...]
    s_full = jnp.concatenate([s_ref[0], s_ref[1]], axis=1)
    h = jnp.maximum(s_full * dis, 0.0)
    u = jnp.dot(h, w_ref[...], preferred_element_type=jnp.float32)
    u = u + b_ref[...]
    g = u * dis
    g_ref[0] = g[:, :HH]
    g_ref[1] = g[:, HH:]


def _t4_body(s_ref, dis_ref, nu_ref, wa1h_ref, wa1nu_ref, ba1_ref, wa2_ref,
             ba2_ref, wo1_ref, bo1_ref, wo2_ref, bo2_ref, out_ref):
    dis = dis_ref[...]
    h3 = jnp.concatenate([s_ref[0], s_ref[1]], axis=1) * dis
    nu = nu_ref[0, 0]
    a = jnp.dot(h3, wa1h_ref[...], preferred_element_type=jnp.float32)
    a = jnp.maximum(a + nu * wa1nu_ref[...] + ba1_ref[...], 0.0)
    att_l = jnp.dot(a, wa2_ref[...], preferred_element_type=jnp.float32)
    att = jax.nn.sigmoid(att_l + ba2_ref[...])
    ah = h3 * att
    z = jnp.dot(ah, wo1_ref[...], preferred_element_type=jnp.float32)
    z = jnp.maximum(z + bo1_ref[...], 0.0)
    o = jnp.dot(z, wo2_ref[...], preferred_element_type=jnp.float32)
    out_ref[...] = jax.nn.sigmoid(o + bo2_ref[...])


def _full(shape):
    return pl.BlockSpec(shape, lambda i: tuple(0 for _ in shape))


_t1 = pl.pallas_call(
    _t1_body,
    grid=(GB,),
    in_specs=[
        pl.BlockSpec((R, D), lambda i: (i, 0)),
        pl.BlockSpec((R, 1), lambda i: (i, 0)),
        pl.BlockSpec((R, 1), lambda i: (i, 0)),
        _full((1, 1)),
        _full((1, 32)),
        _full((1, 32)),
        _full((32, D)),
        _full((1, D)),
        _full((D, H)),
        _full((1, H)),
    ],
    out_specs=[
        pl.BlockSpec((2, R, HH), lambda i: (0, i, 0)),
        pl.BlockSpec((R, 1), lambda i: (i, 0)),
    ],
    out_shape=[
        jax.ShapeDtypeStruct((2, NP, HH), jnp.float32),
        jax.ShapeDtypeStruct((NP, 1), jnp.float32),
    ],
)

_t23 = pl.pallas_call(
    _t23_body,
    grid=(GB,),
    in_specs=[
        pl.BlockSpec((2, R, HH), lambda i: (0, i, 0)),
        pl.BlockSpec((R, 1), lambda i: (i, 0)),
        _full((H, H)),
        _full((1, H)),
    ],
    out_specs=pl.BlockSpec((2, R, HH), lambda i: (0, i, 0)),
    out_shape=jax.ShapeDtypeStruct((2, NP, HH), jnp.float32),
)

_t4 = pl.pallas_call(
    _t4_body,
    grid=(GB,),
    in_specs=[
        pl.BlockSpec((2, R, HH), lambda i: (0, i, 0)),
        pl.BlockSpec((R, 1), lambda i: (i, 0)),
        _full((1, 1)),
        _full((H, HH)),
        _full((1, HH)),
        _full((1, HH)),
        _full((HH, 1)),
        _full((1, 1)),
        _full((H, HH)),
        _full((1, HH)),
        _full((HH, 1)),
        _full((1, 1)),
    ],
    out_specs=pl.BlockSpec((R, 1), lambda i: (i, 0)),
    out_shape=jax.ShapeDtypeStruct((NP, 1), jnp.float32),
)


def kernel(x, edge_index, nu, Wg_a, bg_a, Wg_b, bg_b, W1, b1, W2, b2, W3, b3,
           Wa1, ba1, Wa2, ba2, Wo1, bo1, Wo2, bo2):
    src = edge_index[0].astype(jnp.int32)
    dst = edge_index[1].astype(jnp.int32)
    src2 = jnp.concatenate([src, src + NP])
    dst2 = jnp.concatenate([dst, dst])
    xp = jnp.pad(x, ((0, NP - N), (0, 0)))

    part = _deg_sc(dst)
    d0 = part[0:NP].reshape(NP, 1)
    d1 = part[NP:2 * NP].reshape(NP, 1)

    nu2 = nu.reshape(1, 1)
    g, dis = _t1(xp, d0, d1, nu2, Wg_a, bg_a.reshape(1, 32), Wg_b,
                 bg_b.reshape(1, D), W1, b1.reshape(1, H))

    s = _conv_sc(g.reshape(2 * NP, HH), src2, dst2).reshape(2, NP, HH)
    g = _t23(s, dis, W2, b2.reshape(1, H))
    s = _conv_sc(g.reshape(2 * NP, HH), src2, dst2).reshape(2, NP, HH)
    g = _t23(s, dis, W3, b3.reshape(1, H))
    s = _conv_sc(g.reshape(2 * NP, HH), src2, dst2).reshape(2, NP, HH)

    out = _t4(s, dis, nu2, Wa1[:H], Wa1[H:H + 1], ba1.reshape(1, HH),
              Wa2, ba2.reshape(1, 1), Wo1, bo1.reshape(1, HH),
              Wo2, bo2.reshape(1, 1))
    return out[:N]


# speedup vs baseline: 13.1484x; 1.8014x over previous
import functools

import jax
import jax.numpy as jnp
from jax import lax
from jax.experimental import pallas as pl
from jax.experimental.pallas import tpu as pltpu
from jax.experimental.pallas import tpu_sc as plsc

N = 10000
NP = 10240
E = 320000
D = 128
H = 256
HH = H // 2

NTILE = 16
ROWS_PER_TILE = NP // NTILE
EDGES_PER_TILE = E // NTILE
CH = 80
NWIN = EDGES_PER_TILE // CH
ICH = 4000
IWIN = ICH // CH

DEG_PER_TILE = NP // NTILE
DEG_EDGES_PER_TILE = E // (2 * NTILE)
DEG_NWIN = DEG_EDGES_PER_TILE // CH

R = 320
GB = NP // R

_sc_mesh = plsc.VectorSubcoreMesh(core_axis_name="c", subcore_axis_name="s")


@functools.partial(
    pl.kernel,
    out_type=jax.ShapeDtypeStruct((2 * NP,), jnp.float32),
    mesh=_sc_mesh,
    scratch_types=[
        pltpu.VMEM_SHARED((NP,), jnp.float32),
        pltpu.VMEM((CH,), jnp.int32),
        pltpu.VMEM((CH,), jnp.float32),
        pltpu.VMEM((DEG_PER_TILE,), jnp.float32),
    ],
)
def _deg_sc(dst_ref, part_ref, acc, idxb, ones_b, zbuf):
    c = lax.axis_index("c")
    sid = lax.axis_index("s")

    def fill_z(i, _):
        zbuf[pl.ds(i * 16, 16)] = jnp.zeros((16,), jnp.float32)
        return 0

    lax.fori_loop(0, DEG_PER_TILE // 16, fill_z, 0)

    def fill_o(i, _):
        ones_b[pl.ds(i * 16, 16)] = jnp.ones((16,), jnp.float32)
        return 0

    lax.fori_loop(0, CH // 16, fill_o, 0)

    my0 = pl.multiple_of(sid * DEG_PER_TILE, 8)
    pltpu.sync_copy(zbuf, acc.at[pl.ds(my0, DEG_PER_TILE)])
    plsc.subcore_barrier()

    base = c * (E // 2) + sid * DEG_EDGES_PER_TILE

    def win(w, _):
        off = pl.multiple_of(base + w * CH, 8)
        pltpu.sync_copy(dst_ref.at[pl.ds(off, CH)], idxb)
        pltpu.sync_copy(ones_b, acc.at[idxb], add=True)
        return 0

    lax.fori_loop(0, DEG_NWIN, win, 0)
    plsc.subcore_barrier()
    out0 = pl.multiple_of(c * NP + sid * DEG_PER_TILE, 8)
    pltpu.sync_copy(acc.at[pl.ds(my0, DEG_PER_TILE)],
                    part_ref.at[pl.ds(out0, DEG_PER_TILE)])


@functools.partial(
    pl.kernel,
    out_type=jax.ShapeDtypeStruct((2 * NP, HH), jnp.float32),
    mesh=_sc_mesh,
    scratch_types=[
        pltpu.VMEM_SHARED((NP, HH), jnp.float32),
        pltpu.VMEM((ICH,), jnp.int32),
        pltpu.VMEM((ICH,), jnp.int32),
        pltpu.VMEM((CH, HH), jnp.float32),
        pltpu.VMEM((CH, HH), jnp.float32),
        pltpu.SemaphoreType.DMA,
        pltpu.SemaphoreType.DMA,
    ],
)
def _conv_sc(gcat_ref, src2_ref, dst2_ref, scat_ref, acc, sbuf, dbuf,
             buf0, buf1, sem0, sem1):
    c = lax.axis_index("c")
    sid = lax.axis_index("s")

    r0 = pl.multiple_of(sid * ROWS_PER_TILE, 8)
    grow0 = pl.multiple_of(c * NP + sid * ROWS_PER_TILE, 8)
    ebase = pl.multiple_of(c * E + sid * EDGES_PER_TILE, 8)

    pltpu.sync_copy(gcat_ref.at[pl.ds(grow0, ROWS_PER_TILE)],
                    acc.at[pl.ds(r0, ROWS_PER_TILE)])
    plsc.subcore_barrier()

    def gather(w, buf, sem):
        idx = sbuf.at[pl.ds(pl.multiple_of(w * CH, 8), CH)]
        return pltpu.async_copy(gcat_ref.at[idx], buf, sem)

    def scat_add(w, buf):
        idx = dbuf.at[pl.ds(pl.multiple_of(w * CH, 8), CH)]
        pltpu.sync_copy(buf, acc.at[idx], add=True)

    def wait(buf, sem):
        pltpu.make_async_copy(gcat_ref.at[sbuf.at[pl.ds(0, CH)]],
                              buf, sem).wait()

    def chunk(k, _):
        eoff = pl.multiple_of(ebase + k * ICH, 8)
        pltpu.sync_copy(src2_ref.at[pl.ds(eoff, ICH)], sbuf)
        pltpu.sync_copy(dst2_ref.at[pl.ds(eoff, ICH)], dbuf)
        gather(0, buf0, sem0)

        def win2(i, _):
            w = i * 2
            wait(buf0, sem0)
            gather(w + 1, buf1, sem1)
            scat_add(w, buf0)
            wait(buf1, sem1)

            @pl.when(w + 2 < IWIN)
            def _():
                gather(w + 2, buf0, sem0)

            scat_add(w + 1, buf1)
            return 0

        lax.fori_loop(0, IWIN // 2, win2, 0)
        return 0

    lax.fori_loop(0, EDGES_PER_TILE // ICH, chunk, 0)
    plsc.subcore_barrier()
    pltpu.sync_copy(acc.at[pl.ds(r0, ROWS_PER_TILE)],
                    scat_ref.at[pl.ds(grow0, ROWS_PER_TILE)])


def _t1_body(x_ref, d0_ref, d1_ref, nu_ref, wga_ref, bga_ref, wgb_ref,
             bgb_ref, w1_ref, b1_ref, g_ref, dis_ref):
    nu = nu_ref[0, 0]
    t = jnp.maximum(nu * wga_ref[...] + bga_ref[...], 0.0)
    logits = jnp.dot(t, wgb_ref[...], preferred_element_type=jnp.float32)
    logits = logits + bgb_ref[...]
    m = jnp.max(logits, axis=-1, keepdims=True)
    ex = jnp.exp(logits - m)
    fw = ex / jnp.sum(ex, axis=-1, keepdims=True)
    h0 = x_ref[...] * fw
    u = jnp.dot(h0, w1_ref[...], preferred_element_type=jnp.float32)
    u = u + b1_ref[...]
    deg = d0_ref[...] + d1_ref[...] + 1.0
    dis = lax.rsqrt(jnp.maximum(deg, 1.0))
    dis_ref[...] = dis
    g = u * dis
    g_ref[0] = g[:, :HH]
    g_ref[1] = g[:, HH:]


def _t23_body(s_ref, dis_ref, w_ref, b_ref, g_ref):
    dis = dis_ref[...]
    s_full = jnp.concatenate([s_ref[0], s_ref[1]], axis=1)
    h = jnp.maximum(s_full * dis, 0.0)
    u = jnp.dot(h, w_ref[...], preferred_element_type=jnp.float32)
    u = u + b_ref[...]
    g = u * dis
    g_ref[0] = g[:, :HH]
    g_ref[1] = g[:, HH:]


def _t4_body(s_ref, dis_ref, nu_ref, wa1h_ref, wa1nu_ref, ba1_ref, wa2_ref,
             ba2_ref, wo1_ref, bo1_ref, wo2_ref, bo2_ref, out_ref):
    dis = dis_ref[...]
    h3 = jnp.concatenate([s_ref[0], s_ref[1]], axis=1) * dis
    nu = nu_ref[0, 0]
    a = jnp.dot(h3, wa1h_ref[...], preferred_element_type=jnp.float32)
    a = jnp.maximum(a + nu * wa1nu_ref[...] + ba1_ref[...], 0.0)
    att_l = jnp.dot(a, wa2_ref[...], preferred_element_type=jnp.float32)
    att = jax.nn.sigmoid(att_l + ba2_ref[...])
    ah = h3 * att
    z = jnp.dot(ah, wo1_ref[...], preferred_element_type=jnp.float32)
    z = jnp.maximum(z + bo1_ref[...], 0.0)
    o = jnp.dot(z, wo2_ref[...], preferred_element_type=jnp.float32)
    out_ref[...] = jax.nn.sigmoid(o + bo2_ref[...])


def _full(shape):
    return pl.BlockSpec(shape, lambda i: tuple(0 for _ in shape))


_t1 = pl.pallas_call(
    _t1_body,
    grid=(GB,),
    in_specs=[
        pl.BlockSpec((R, D), lambda i: (i, 0)),
        pl.BlockSpec((R, 1), lambda i: (i, 0)),
        pl.BlockSpec((R, 1), lambda i: (i, 0)),
        _full((1, 1)),
        _full((1, 32)),
        _full((1, 32)),
        _full((32, D)),
        _full((1, D)),
        _full((D, H)),
        _full((1, H)),
    ],
    out_specs=[
        pl.BlockSpec((2, R, HH), lambda i: (0, i, 0)),
        pl.BlockSpec((R, 1), lambda i: (i, 0)),
    ],
    out_shape=[
        jax.ShapeDtypeStruct((2, NP, HH), jnp.float32),
        jax.ShapeDtypeStruct((NP, 1), jnp.float32),
    ],
)

_t23 = pl.pallas_call(
    _t23_body,
    grid=(GB,),
    in_specs=[
        pl.BlockSpec((2, R, HH), lambda i: (0, i, 0)),
        pl.BlockSpec((R, 1), lambda i: (i, 0)),
        _full((H, H)),
        _full((1, H)),
    ],
    out_specs=pl.BlockSpec((2, R, HH), lambda i: (0, i, 0)),
    out_shape=jax.ShapeDtypeStruct((2, NP, HH), jnp.float32),
)

_t4 = pl.pallas_call(
    _t4_body,
    grid=(GB,),
    in_specs=[
        pl.BlockSpec((2, R, HH), lambda i: (0, i, 0)),
        pl.BlockSpec((R, 1), lambda i: (i, 0)),
        _full((1, 1)),
        _full((H, HH)),
        _full((1, HH)),
        _full((1, HH)),
        _full((HH, 1)),
        _full((1, 1)),
        _full((H, HH)),
        _full((1, HH)),
        _full((HH, 1)),
        _full((1, 1)),
    ],
    out_specs=pl.BlockSpec((R, 1), lambda i: (i, 0)),
    out_shape=jax.ShapeDtypeStruct((NP, 1), jnp.float32),
)


def kernel(x, edge_index, nu, Wg_a, bg_a, Wg_b, bg_b, W1, b1, W2, b2, W3, b3,
           Wa1, ba1, Wa2, ba2, Wo1, bo1, Wo2, bo2):
    src = edge_index[0].astype(jnp.int32)
    dst = edge_index[1].astype(jnp.int32)
    src2 = jnp.concatenate([src, src + NP])
    dst2 = jnp.concatenate([dst, dst])
    xp = jnp.pad(x, ((0, NP - N), (0, 0)))

    part = _deg_sc(dst)
    d0 = part[0:NP].reshape(NP, 1)
    d1 = part[NP:2 * NP].reshape(NP, 1)

    nu2 = nu.reshape(1, 1)
    g, dis = _t1(xp, d0, d1, nu2, Wg_a, bg_a.reshape(1, 32), Wg_b,
                 bg_b.reshape(1, D), W1, b1.reshape(1, H))

    s = _conv_sc(g.reshape(2 * NP, HH), src2, dst2).reshape(2, NP, HH)
    g = _t23(s, dis, W2, b2.reshape(1, H))
    s = _conv_sc(g.reshape(2 * NP, HH), src2, dst2).reshape(2, NP, HH)
    g = _t23(s, dis, W3, b3.reshape(1, H))
    s = _conv_sc(g.reshape(2 * NP, HH), src2, dst2).reshape(2, NP, HH)

    out = _t4(s, dis, nu2, Wa1[:H], Wa1[H:H + 1], ba1.reshape(1, HH),
              Wa2, ba2.reshape(1, 1), Wo1, bo1.reshape(1, HH),
              Wo2, bo2.reshape(1, 1))
    return out[:N]


# speedup vs baseline: 13.3460x; 1.0150x over previous
import functools

import jax
import jax.numpy as jnp
from jax import lax
from jax.experimental import pallas as pl
from jax.experimental.pallas import tpu as pltpu
from jax.experimental.pallas import tpu_sc as plsc

N = 10000
NP = 10240
E = 320000
D = 128
H = 256
HH = H // 2

NTILE = 16
ROWS_PER_TILE = NP // NTILE
EDGES_PER_TILE = E // NTILE
CH = 80
NWIN = EDGES_PER_TILE // CH
ICH = 4000
IWIN = ICH // CH

DEG_PER_TILE = NP // NTILE
DEG_EDGES_PER_TILE = E // (2 * NTILE)
DEG_NWIN = DEG_EDGES_PER_TILE // CH

R = 320
GB = NP // R

_sc_mesh = plsc.VectorSubcoreMesh(core_axis_name="c", subcore_axis_name="s")


@functools.partial(
    pl.kernel,
    out_type=jax.ShapeDtypeStruct((2 * NP,), jnp.float32),
    mesh=_sc_mesh,
    scratch_types=[
        pltpu.VMEM_SHARED((NP,), jnp.float32),
        pltpu.VMEM((CH,), jnp.int32),
        pltpu.VMEM((CH,), jnp.float32),
        pltpu.VMEM((DEG_PER_TILE,), jnp.float32),
    ],
)
def _deg_sc(dst_ref, part_ref, acc, idxb, ones_b, zbuf):
    c = lax.axis_index("c")
    sid = lax.axis_index("s")

    def fill_z(i, _):
        zbuf[pl.ds(i * 16, 16)] = jnp.zeros((16,), jnp.float32)
        return 0

    lax.fori_loop(0, DEG_PER_TILE // 16, fill_z, 0)

    def fill_o(i, _):
        ones_b[pl.ds(i * 16, 16)] = jnp.ones((16,), jnp.float32)
        return 0

    lax.fori_loop(0, CH // 16, fill_o, 0)

    my0 = pl.multiple_of(sid * DEG_PER_TILE, 8)
    pltpu.sync_copy(zbuf, acc.at[pl.ds(my0, DEG_PER_TILE)])
    plsc.subcore_barrier()

    base = c * (E // 2) + sid * DEG_EDGES_PER_TILE

    def win(w, _):
        off = pl.multiple_of(base + w * CH, 8)
        pltpu.sync_copy(dst_ref.at[pl.ds(off, CH)], idxb)
        pltpu.sync_copy(ones_b, acc.at[idxb], add=True)
        return 0

    lax.fori_loop(0, DEG_NWIN, win, 0)
    plsc.subcore_barrier()
    out0 = pl.multiple_of(c * NP + sid * DEG_PER_TILE, 8)
    pltpu.sync_copy(acc.at[pl.ds(my0, DEG_PER_TILE)],
                    part_ref.at[pl.ds(out0, DEG_PER_TILE)])


@functools.partial(
    pl.kernel,
    out_type=jax.ShapeDtypeStruct((2 * NP, HH), jnp.float32),
    mesh=_sc_mesh,
    scratch_types=[
        pltpu.VMEM_SHARED((NP, HH), jnp.float32),
        pltpu.VMEM((ICH,), jnp.int32),
        pltpu.VMEM((ICH,), jnp.int32),
        pltpu.VMEM((CH, HH), jnp.float32),
        pltpu.VMEM((CH, HH), jnp.float32),
        pltpu.SemaphoreType.DMA,
        pltpu.SemaphoreType.DMA,
        pltpu.SemaphoreType.DMA,
        pltpu.SemaphoreType.DMA,
    ],
)
def _conv_sc(gcat_ref, src2_ref, dst2_ref, scat_ref, acc, sbuf, dbuf,
             buf0, buf1, sem0, sem1, ssem0, ssem1):
    c = lax.axis_index("c")
    sid = lax.axis_index("s")

    r0 = pl.multiple_of(sid * ROWS_PER_TILE, 8)
    grow0 = pl.multiple_of(c * NP + sid * ROWS_PER_TILE, 8)
    ebase = pl.multiple_of(c * E + sid * EDGES_PER_TILE, 8)

    pltpu.sync_copy(gcat_ref.at[pl.ds(grow0, ROWS_PER_TILE)],
                    acc.at[pl.ds(r0, ROWS_PER_TILE)])
    plsc.subcore_barrier()

    def gather(w, buf, sem):
        idx = sbuf.at[pl.ds(pl.multiple_of(w * CH, 8), CH)]
        pltpu.async_copy(gcat_ref.at[idx], buf, sem)

    def scat_add(w, buf, sem):
        idx = dbuf.at[pl.ds(pl.multiple_of(w * CH, 8), CH)]
        pltpu.async_copy(buf, acc.at[idx], sem, add=True)

    def gwait(buf, sem):
        pltpu.make_async_copy(gcat_ref.at[sbuf.at[pl.ds(0, CH)]],
                              buf, sem).wait()

    def swait(buf, sem):
        pltpu.make_async_copy(buf, acc.at[dbuf.at[pl.ds(0, CH)]],
                              sem).wait()

    def chunk(k, _):
        eoff = pl.multiple_of(ebase + k * ICH, 8)
        pltpu.sync_copy(src2_ref.at[pl.ds(eoff, ICH)], sbuf)
        pltpu.sync_copy(dst2_ref.at[pl.ds(eoff, ICH)], dbuf)
        gather(0, buf0, sem0)
        gather(1, buf1, sem1)

        def win2(i, _):
            w = i * 2
            gwait(buf0, sem0)
            scat_add(w, buf0, ssem0)
            gwait(buf1, sem1)
            scat_add(w + 1, buf1, ssem1)

            @pl.when(w + 2 < IWIN)
            def _():
                swait(buf0, ssem0)
                gather(w + 2, buf0, sem0)
                swait(buf1, ssem1)
                gather(w + 3, buf1, sem1)

            return 0

        lax.fori_loop(0, IWIN // 2, win2, 0)
        swait(buf0, ssem0)
        swait(buf1, ssem1)
        return 0

    lax.fori_loop(0, EDGES_PER_TILE // ICH, chunk, 0)
    plsc.subcore_barrier()
    pltpu.sync_copy(acc.at[pl.ds(r0, ROWS_PER_TILE)],
                    scat_ref.at[pl.ds(grow0, ROWS_PER_TILE)])


def _t1_body(x_ref, d0_ref, d1_ref, nu_ref, wga_ref, bga_ref, wgb_ref,
             bgb_ref, w1_ref, b1_ref, g_ref, dis_ref):
    nu = nu_ref[0, 0]
    t = jnp.maximum(nu * wga_ref[...] + bga_ref[...], 0.0)
    logits = jnp.dot(t, wgb_ref[...], preferred_element_type=jnp.float32)
    logits = logits + bgb_ref[...]
    m = jnp.max(logits, axis=-1, keepdims=True)
    ex = jnp.exp(logits - m)
    fw = ex / jnp.sum(ex, axis=-1, keepdims=True)
    h0 = x_ref[...] * fw
    u = jnp.dot(h0, w1_ref[...], preferred_element_type=jnp.float32)
    u = u + b1_ref[...]
    deg = d0_ref[...] + d1_ref[...] + 1.0
    dis = lax.rsqrt(jnp.maximum(deg, 1.0))
    dis_ref[...] = dis
    g = u * dis
    g_ref[0] = g[:, :HH]
    g_ref[1] = g[:, HH:]


def _t23_body(s_ref, dis_ref, w_ref, b_ref, g_ref):
    dis = dis_ref[...]
    s_full = jnp.concatenate([s_ref[0], s_ref[1]], axis=1)
    h = jnp.maximum(s_full * dis, 0.0)
    u = jnp.dot(h, w_ref[...], preferred_element_type=jnp.float32)
    u = u + b_ref[...]
    g = u * dis
    g_ref[0] = g[:, :HH]
    g_ref[1] = g[:, HH:]


def _t4_body(s_ref, dis_ref, nu_ref, wa1h_ref, wa1nu_ref, ba1_ref, wa2_ref,
             ba2_ref, wo1_ref, bo1_ref, wo2_ref, bo2_ref, out_ref):
    dis = dis_ref[...]
    h3 = jnp.concatenate([s_ref[0], s_ref[1]], axis=1) * dis
    nu = nu_ref[0, 0]
    a = jnp.dot(h3, wa1h_ref[...], preferred_element_type=jnp.float32)
    a = jnp.maximum(a + nu * wa1nu_ref[...] + ba1_ref[...], 0.0)
    att_l = jnp.dot(a, wa2_ref[...], preferred_element_type=jnp.float32)
    att = jax.nn.sigmoid(att_l + ba2_ref[...])
    ah = h3 * att
    z = jnp.dot(ah, wo1_ref[...], preferred_element_type=jnp.float32)
    z = jnp.maximum(z + bo1_ref[...], 0.0)
    o = jnp.dot(z, wo2_ref[...], preferred_element_type=jnp.float32)
    out_ref[...] = jax.nn.sigmoid(o + bo2_ref[...])


def _full(shape):
    return pl.BlockSpec(shape, lambda i: tuple(0 for _ in shape))


_t1 = pl.pallas_call(
    _t1_body,
    grid=(GB,),
    in_specs=[
        pl.BlockSpec((R, D), lambda i: (i, 0)),
        pl.BlockSpec((R, 1), lambda i: (i, 0)),
        pl.BlockSpec((R, 1), lambda i: (i, 0)),
        _full((1, 1)),
        _full((1, 32)),
        _full((1, 32)),
        _full((32, D)),
        _full((1, D)),
        _full((D, H)),
        _full((1, H)),
    ],
    out_specs=[
        pl.BlockSpec((2, R, HH), lambda i: (0, i, 0)),
        pl.BlockSpec((R, 1), lambda i: (i, 0)),
    ],
    out_shape=[
        jax.ShapeDtypeStruct((2, NP, HH), jnp.float32),
        jax.ShapeDtypeStruct((NP, 1), jnp.float32),
    ],
)

_t23 = pl.pallas_call(
    _t23_body,
    grid=(GB,),
    in_specs=[
        pl.BlockSpec((2, R, HH), lambda i: (0, i, 0)),
        pl.BlockSpec((R, 1), lambda i: (i, 0)),
        _full((H, H)),
        _full((1, H)),
    ],
    out_specs=pl.BlockSpec((2, R, HH), lambda i: (0, i, 0)),
    out_shape=jax.ShapeDtypeStruct((2, NP, HH), jnp.float32),
)

_t4 = pl.pallas_call(
    _t4_body,
    grid=(GB,),
    in_specs=[
        pl.BlockSpec((2, R, HH), lambda i: (0, i, 0)),
        pl.BlockSpec((R, 1), lambda i: (i, 0)),
        _full((1, 1)),
        _full((H, HH)),
        _full((1, HH)),
        _full((1, HH)),
        _full((HH, 1)),
        _full((1, 1)),
        _full((H, HH)),
        _full((1, HH)),
        _full((HH, 1)),
        _full((1, 1)),
    ],
    out_specs=pl.BlockSpec((R, 1), lambda i: (i, 0)),
    out_shape=jax.ShapeDtypeStruct((NP, 1), jnp.float32),
)


def kernel(x, edge_index, nu, Wg_a, bg_a, Wg_b, bg_b, W1, b1, W2, b2, W3, b3,
           Wa1, ba1, Wa2, ba2, Wo1, bo1, Wo2, bo2):
    src = edge_index[0].astype(jnp.int32)
    dst = edge_index[1].astype(jnp.int32)
    src2 = jnp.concatenate([src, src + NP])
    dst2 = jnp.concatenate([dst, dst])
    xp = jnp.pad(x, ((0, NP - N), (0, 0)))

    part = _deg_sc(dst)
    d0 = part[0:NP].reshape(NP, 1)
    d1 = part[NP:2 * NP].reshape(NP, 1)

    nu2 = nu.reshape(1, 1)
    g, dis = _t1(xp, d0, d1, nu2, Wg_a, bg_a.reshape(1, 32), Wg_b,
                 bg_b.reshape(1, D), W1, b1.reshape(1, H))

    s = _conv_sc(g.reshape(2 * NP, HH), src2, dst2).reshape(2, NP, HH)
    g = _t23(s, dis, W2, b2.reshape(1, H))
    s = _conv_sc(g.reshape(2 * NP, HH), src2, dst2).reshape(2, NP, HH)
    g = _t23(s, dis, W3, b3.reshape(1, H))
    s = _conv_sc(g.reshape(2 * NP, HH), src2, dst2).reshape(2, NP, HH)

    out = _t4(s, dis, nu2, Wa1[:H], Wa1[H:H + 1], ba1.reshape(1, HH),
              Wa2, ba2.reshape(1, 1), Wo1, bo1.reshape(1, HH),
              Wo2, bo2.reshape(1, 1))
    return out[:N]


# speedup vs baseline: 14.1520x; 1.0604x over previous
import functools

import jax
import jax.numpy as jnp
from jax import lax
from jax.experimental import pallas as pl
from jax.experimental.pallas import tpu as pltpu
from jax.experimental.pallas import tpu_sc as plsc

N = 10000
NP = 10240
E = 320000
D = 128
H = 256
HH = H // 2

NTILE = 16
ROWS_PER_TILE = NP // NTILE
CH = 128
WPT = 160
EPT = CH * WPT
CH_PAD = EPT - E // NTILE
ICH = 5120
IWIN = ICH // CH

DEG_PER_TILE = NP // NTILE
DEG_EDGES_PER_TILE = E // (2 * NTILE)
DEG_NWIN = DEG_EDGES_PER_TILE // CH

R = 320
GB = NP // R

_sc_mesh = plsc.VectorSubcoreMesh(core_axis_name="c", subcore_axis_name="s")


@functools.partial(
    pl.kernel,
    out_type=jax.ShapeDtypeStruct((2 * NP,), jnp.float32),
    mesh=_sc_mesh,
    scratch_types=[
        pltpu.VMEM_SHARED((NP,), jnp.float32),
        pltpu.VMEM((CH,), jnp.int32),
        pltpu.VMEM((CH,), jnp.float32),
        pltpu.VMEM((DEG_PER_TILE,), jnp.float32),
    ],
)
def _deg_sc(dst_ref, part_ref, acc, idxb, ones_b, zbuf):
    c = lax.axis_index("c")
    sid = lax.axis_index("s")

    def fill_z(i, _):
        zbuf[pl.ds(i * 16, 16)] = jnp.zeros((16,), jnp.float32)
        return 0

    lax.fori_loop(0, DEG_PER_TILE // 16, fill_z, 0)

    def fill_o(i, _):
        ones_b[pl.ds(i * 16, 16)] = jnp.ones((16,), jnp.float32)
        return 0

    lax.fori_loop(0, CH // 16, fill_o, 0)

    my0 = pl.multiple_of(sid * DEG_PER_TILE, 8)
    pltpu.sync_copy(zbuf, acc.at[pl.ds(my0, DEG_PER_TILE)])
    plsc.subcore_barrier()

    base = c * (E // 2) + sid * DEG_EDGES_PER_TILE

    def win(w, _):
        off = pl.multiple_of(base + w * CH, 8)
        pltpu.sync_copy(dst_ref.at[pl.ds(off, CH)], idxb)
        pltpu.sync_copy(ones_b, acc.at[idxb], add=True)
        return 0

    lax.fori_loop(0, DEG_NWIN, win, 0)
    plsc.subcore_barrier()
    out0 = pl.multiple_of(c * NP + sid * DEG_PER_TILE, 8)
    pltpu.sync_copy(acc.at[pl.ds(my0, DEG_PER_TILE)],
                    part_ref.at[pl.ds(out0, DEG_PER_TILE)])


@functools.partial(
    pl.kernel,
    out_type=jax.ShapeDtypeStruct((2 * NP, HH), jnp.float32),
    mesh=_sc_mesh,
    scratch_types=[
        pltpu.VMEM_SHARED((NP, HH), jnp.float32),
        pltpu.VMEM((ICH,), jnp.int32),
        pltpu.VMEM((ICH,), jnp.int32),
        pltpu.VMEM((CH, HH), jnp.float32),
        pltpu.VMEM((CH, HH), jnp.float32),
        pltpu.SemaphoreType.DMA,
        pltpu.SemaphoreType.DMA,
        pltpu.SemaphoreType.DMA,
        pltpu.SemaphoreType.DMA,
    ],
)
def _conv_sc(gcat_ref, src2_ref, dst2_ref, scat_ref, acc, sbuf, dbuf,
             buf0, buf1, sem0, sem1, ssem0, ssem1):
    c = lax.axis_index("c")
    sid = lax.axis_index("s")

    r0 = pl.multiple_of(sid * ROWS_PER_TILE, 8)
    grow0 = pl.multiple_of(c * NP + sid * ROWS_PER_TILE, 8)
    ebase = pl.multiple_of((c * NTILE + sid) * EPT, 8)

    pltpu.sync_copy(gcat_ref.at[pl.ds(grow0, ROWS_PER_TILE)],
                    acc.at[pl.ds(r0, ROWS_PER_TILE)])
    plsc.subcore_barrier()

    def gather(w, buf, sem):
        idx = sbuf.at[pl.ds(pl.multiple_of(w * CH, 8), CH)]
        pltpu.async_copy(gcat_ref.at[idx], buf, sem)

    def scat_add(w, buf, sem):
        idx = dbuf.at[pl.ds(pl.multiple_of(w * CH, 8), CH)]
        pltpu.async_copy(buf, acc.at[idx], sem, add=True)

    def gwait(buf, sem):
        pltpu.make_async_copy(gcat_ref.at[sbuf.at[pl.ds(0, CH)]],
                              buf, sem).wait()

    def swait(buf, sem):
        pltpu.make_async_copy(buf, acc.at[dbuf.at[pl.ds(0, CH)]],
                              sem).wait()

    def chunk(k, _):
        eoff = pl.multiple_of(ebase + k * ICH, 8)
        pltpu.sync_copy(src2_ref.at[pl.ds(eoff, ICH)], sbuf)
        pltpu.sync_copy(dst2_ref.at[pl.ds(eoff, ICH)], dbuf)
        gather(0, buf0, sem0)
        gather(1, buf1, sem1)

        def win2(i, _):
            w = i * 2
            gwait(buf0, sem0)
            scat_add(w, buf0, ssem0)
            gwait(buf1, sem1)
            scat_add(w + 1, buf1, ssem1)

            @pl.when(w + 2 < IWIN)
            def _():
                swait(buf0, ssem0)
                gather(w + 2, buf0, sem0)
                swait(buf1, ssem1)
                gather(w + 3, buf1, sem1)

            return 0

        lax.fori_loop(0, IWIN // 2, win2, 0)
        swait(buf0, ssem0)
        swait(buf1, ssem1)
        return 0

    lax.fori_loop(0, EPT // ICH, chunk, 0)
    plsc.subcore_barrier()
    pltpu.sync_copy(acc.at[pl.ds(r0, ROWS_PER_TILE)],
                    scat_ref.at[pl.ds(grow0, ROWS_PER_TILE)])


def _t1_body(x_ref, d0_ref, d1_ref, nu_ref, wga_ref, bga_ref, wgb_ref,
             bgb_ref, w1_ref, b1_ref, g_ref, dis_ref):
    nu = nu_ref[0, 0]
    t = jnp.maximum(nu * wga_ref[...] + bga_ref[...], 0.0)
    logits = jnp.dot(t, wgb_ref[...], preferred_element_type=jnp.float32)
    logits = logits + bgb_ref[...]
    m = jnp.max(logits, axis=-1, keepdims=True)
    ex = jnp.exp(logits - m)
    fw = ex / jnp.sum(ex, axis=-1, keepdims=True)
    h0 = x_ref[...] * fw
    u = jnp.dot(h0, w1_ref[...], preferred_element_type=jnp.float32)
    u = u + b1_ref[...]
    deg = d0_ref[...] + d1_ref[...] + 1.0
    dis = lax.rsqrt(jnp.maximum(deg, 1.0))
    dis_ref[...] = dis
    g = u * dis
    g_ref[0] = g[:, :HH]
    g_ref[1] = g[:, HH:]


def _t23_body(s_ref, dis_ref, w_ref, b_ref, g_ref):
    dis = dis_ref[...]
    s_full = jnp.concatenate([s_ref[0], s_ref[1]], axis=1)
    h = jnp.maximum(s_full * dis, 0.0)
    u = jnp.dot(h, w_ref[...], preferred_element_type=jnp.float32)
    u = u + b_ref[...]
    g = u * dis
    g_ref[0] = g[:, :HH]
    g_ref[1] = g[:, HH:]


def _t4_body(s_ref, dis_ref, nu_ref, wa1h_ref, wa1nu_ref, ba1_ref, wa2_ref,
             ba2_ref, wo1_ref, bo1_ref, wo2_ref, bo2_ref, out_ref):
    dis = dis_ref[...]
    h3 = jnp.concatenate([s_ref[0], s_ref[1]], axis=1) * dis
    nu = nu_ref[0, 0]
    a = jnp.dot(h3, wa1h_ref[...], preferred_element_type=jnp.float32)
    a = jnp.maximum(a + nu * wa1nu_ref[...] + ba1_ref[...], 0.0)
    att_l = jnp.dot(a, wa2_ref[...], preferred_element_type=jnp.float32)
    att = jax.nn.sigmoid(att_l + ba2_ref[...])
    ah = h3 * att
    z = jnp.dot(ah, wo1_ref[...], preferred_element_type=jnp.float32)
    z = jnp.maximum(z + bo1_ref[...], 0.0)
    o = jnp.dot(z, wo2_ref[...], preferred_element_type=jnp.float32)
    out_ref[...] = jax.nn.sigmoid(o + bo2_ref[...])


def _full(shape):
    return pl.BlockSpec(shape, lambda i: tuple(0 for _ in shape))


_t1 = pl.pallas_call(
    _t1_body,
    grid=(GB,),
    in_specs=[
        pl.BlockSpec((R, D), lambda i: (i, 0)),
        pl.BlockSpec((R, 1), lambda i: (i, 0)),
        pl.BlockSpec((R, 1), lambda i: (i, 0)),
        _full((1, 1)),
        _full((1, 32)),
        _full((1, 32)),
        _full((32, D)),
        _full((1, D)),
        _full((D, H)),
        _full((1, H)),
    ],
    out_specs=[
        pl.BlockSpec((2, R, HH), lambda i: (0, i, 0)),
        pl.BlockSpec((R, 1), lambda i: (i, 0)),
    ],
    out_shape=[
        jax.ShapeDtypeStruct((2, NP, HH), jnp.float32),
        jax.ShapeDtypeStruct((NP, 1), jnp.float32),
    ],
)

_t23 = pl.pallas_call(
    _t23_body,
    grid=(GB,),
    in_specs=[
        pl.BlockSpec((2, R, HH), lambda i: (0, i, 0)),
        pl.BlockSpec((R, 1), lambda i: (i, 0)),
        _full((H, H)),
        _full((1, H)),
    ],
    out_specs=pl.BlockSpec((2, R, HH), lambda i: (0, i, 0)),
    out_shape=jax.ShapeDtypeStruct((2, NP, HH), jnp.float32),
)

_t4 = pl.pallas_call(
    _t4_body,
    grid=(GB,),
    in_specs=[
        pl.BlockSpec((2, R, HH), lambda i: (0, i, 0)),
        pl.BlockSpec((R, 1), lambda i: (i, 0)),
        _full((1, 1)),
        _full((H, HH)),
        _full((1, HH)),
        _full((1, HH)),
        _full((HH, 1)),
        _full((1, 1)),
        _full((H, HH)),
        _full((1, HH)),
        _full((HH, 1)),
        _full((1, 1)),
    ],
    out_specs=pl.BlockSpec((R, 1), lambda i: (i, 0)),
    out_shape=jax.ShapeDtypeStruct((NP, 1), jnp.float32),
)


def kernel(x, edge_index, nu, Wg_a, bg_a, Wg_b, bg_b, W1, b1, W2, b2, W3, b3,
           Wa1, ba1, Wa2, ba2, Wo1, bo1, Wo2, bo2):
    src = edge_index[0].astype(jnp.int32)
    dst = edge_index[1].astype(jnp.int32)
    src3 = jnp.concatenate([src, src + NP]).reshape(2, NTILE, E // NTILE)
    dst3 = jnp.concatenate([dst, dst]).reshape(2, NTILE, E // NTILE)
    pad_s = jnp.broadcast_to(jnp.arange(CH_PAD, dtype=jnp.int32) % N,
                             (2, NTILE, CH_PAD))
    pad_d = jnp.broadcast_to(
        N + jnp.arange(CH_PAD, dtype=jnp.int32) % (NP - N),
        (2, NTILE, CH_PAD))
    src2 = jnp.concatenate([src3, pad_s], axis=2).reshape(-1)
    dst2 = jnp.concatenate([dst3, pad_d], axis=2).reshape(-1)
    xp = jnp.pad(x, ((0, NP - N), (0, 0)))

    part = _deg_sc(dst)
    d0 = part[0:NP].reshape(NP, 1)
    d1 = part[NP:2 * NP].reshape(NP, 1)

    nu2 = nu.reshape(1, 1)
    g, dis = _t1(xp, d0, d1, nu2, Wg_a, bg_a.reshape(1, 32), Wg_b,
                 bg_b.reshape(1, D), W1, b1.reshape(1, H))

    s = _conv_sc(g.reshape(2 * NP, HH), src2, dst2).reshape(2, NP, HH)
    g = _t23(s, dis, W2, b2.reshape(1, H))
    s = _conv_sc(g.reshape(2 * NP, HH), src2, dst2).reshape(2, NP, HH)
    g = _t23(s, dis, W3, b3.reshape(1, H))
    s = _conv_sc(g.reshape(2 * NP, HH), src2, dst2).reshape(2, NP, HH)

    out = _t4(s, dis, nu2, Wa1[:H], Wa1[H:H + 1], ba1.reshape(1, HH),
              Wa2, ba2.reshape(1, 1), Wo1, bo1.reshape(1, HH),
              Wo2, bo2.reshape(1, 1))
    return out[:N]


# speedup vs baseline: 14.3731x; 1.0156x over previous
import functools

import jax
import jax.numpy as jnp
from jax import lax
from jax.experimental import pallas as pl
from jax.experimental.pallas import tpu as pltpu
from jax.experimental.pallas import tpu_sc as plsc

N = 10000
NP = 10240
E = 320000
D = 128
H = 256
HH = H // 2

NTILE = 16
ROWS_PER_TILE = NP // NTILE
CH = 128
WPT = 160
EPT = CH * WPT
CH_PAD = EPT - E // NTILE
ICH = 5120
IWIN = ICH // CH

DEG_PER_TILE = NP // NTILE
DEG_EDGES_PER_TILE = E // (2 * NTILE)
DEG_NWIN = DEG_EDGES_PER_TILE // CH
DEG_TAIL = DEG_EDGES_PER_TILE - DEG_NWIN * CH

R = 320
GB = NP // R

_sc_mesh = plsc.VectorSubcoreMesh(core_axis_name="c", subcore_axis_name="s")


@functools.partial(
    pl.kernel,
    out_type=jax.ShapeDtypeStruct((2 * NP,), jnp.float32),
    mesh=_sc_mesh,
    scratch_types=[
        pltpu.VMEM_SHARED((NP,), jnp.float32),
        pltpu.VMEM((DEG_EDGES_PER_TILE,), jnp.int32),
        pltpu.VMEM((CH,), jnp.float32),
        pltpu.VMEM((DEG_PER_TILE,), jnp.float32),
        pltpu.SemaphoreType.DMA,
    ],
)
def _deg_sc(dst_ref, part_ref, acc, idxb, ones_b, zbuf, ssem):
    c = lax.axis_index("c")
    sid = lax.axis_index("s")

    def fill_z(i, _):
        zbuf[pl.ds(i * 16, 16)] = jnp.zeros((16,), jnp.float32)
        return 0

    lax.fori_loop(0, DEG_PER_TILE // 16, fill_z, 0)

    def fill_o(i, _):
        ones_b[pl.ds(i * 16, 16)] = jnp.ones((16,), jnp.float32)
        return 0

    lax.fori_loop(0, CH // 16, fill_o, 0)

    base = pl.multiple_of(c * (E // 2) + sid * DEG_EDGES_PER_TILE, 8)
    pltpu.sync_copy(dst_ref.at[pl.ds(base, DEG_EDGES_PER_TILE)], idxb)

    my0 = pl.multiple_of(sid * DEG_PER_TILE, 8)
    pltpu.sync_copy(zbuf, acc.at[pl.ds(my0, DEG_PER_TILE)])
    plsc.subcore_barrier()

    def win(w, _):
        idx = idxb.at[pl.ds(pl.multiple_of(w * CH, 8), CH)]
        pltpu.async_copy(ones_b, acc.at[idx], ssem, add=True)
        return 0

    lax.fori_loop(0, DEG_NWIN, win, 0)
    tidx = idxb.at[pl.ds(pl.multiple_of(DEG_NWIN * CH, 8), DEG_TAIL)]
    pltpu.async_copy(ones_b.at[pl.ds(0, DEG_TAIL)], acc.at[tidx], ssem,
                     add=True)

    def drain(w, _):
        idx = idxb.at[pl.ds(0, CH)]
        pltpu.make_async_copy(ones_b, acc.at[idx], ssem).wait()
        return 0

    lax.fori_loop(0, DEG_NWIN, drain, 0)
    pltpu.make_async_copy(ones_b.at[pl.ds(0, DEG_TAIL)],
                          acc.at[idxb.at[pl.ds(0, DEG_TAIL)]], ssem).wait()
    plsc.subcore_barrier()
    out0 = pl.multiple_of(c * NP + sid * DEG_PER_TILE, 8)
    pltpu.sync_copy(acc.at[pl.ds(my0, DEG_PER_TILE)],
                    part_ref.at[pl.ds(out0, DEG_PER_TILE)])


@functools.partial(
    pl.kernel,
    out_type=jax.ShapeDtypeStruct((2 * NP, HH), jnp.float32),
    mesh=_sc_mesh,
    scratch_types=[
        pltpu.VMEM_SHARED((NP, HH), jnp.float32),
        pltpu.VMEM((ICH,), jnp.int32),
        pltpu.VMEM((ICH,), jnp.int32),
        pltpu.VMEM((CH, HH), jnp.float32),
        pltpu.VMEM((CH, HH), jnp.float32),
        pltpu.SemaphoreType.DMA,
        pltpu.SemaphoreType.DMA,
        pltpu.SemaphoreType.DMA,
        pltpu.SemaphoreType.DMA,
    ],
)
def _conv_sc(gcat_ref, src2_ref, dst2_ref, scat_ref, acc, sbuf, dbuf,
             buf0, buf1, sem0, sem1, ssem0, ssem1):
    c = lax.axis_index("c")
    sid = lax.axis_index("s")

    r0 = pl.multiple_of(sid * ROWS_PER_TILE, 8)
    grow0 = pl.multiple_of(c * NP + sid * ROWS_PER_TILE, 8)
    ebase = pl.multiple_of((c * NTILE + sid) * EPT, 8)

    pltpu.sync_copy(gcat_ref.at[pl.ds(grow0, ROWS_PER_TILE)],
                    acc.at[pl.ds(r0, ROWS_PER_TILE)])
    plsc.subcore_barrier()

    def gather(w, buf, sem):
        idx = sbuf.at[pl.ds(pl.multiple_of(w * CH, 8), CH)]
        pltpu.async_copy(gcat_ref.at[idx], buf, sem)

    def scat_add(w, buf, sem):
        idx = dbuf.at[pl.ds(pl.multiple_of(w * CH, 8), CH)]
        pltpu.async_copy(buf, acc.at[idx], sem, add=True)

    def gwait(buf, sem):
        pltpu.make_async_copy(gcat_ref.at[sbuf.at[pl.ds(0, CH)]],
                              buf, sem).wait()

    def swait(buf, sem):
        pltpu.make_async_copy(buf, acc.at[dbuf.at[pl.ds(0, CH)]],
                              sem).wait()

    def chunk(k, _):
        eoff = pl.multiple_of(ebase + k * ICH, 8)
        pltpu.sync_copy(src2_ref.at[pl.ds(eoff, ICH)], sbuf)
        pltpu.sync_copy(dst2_ref.at[pl.ds(eoff, ICH)], dbuf)
        gather(0, buf0, sem0)
        gather(1, buf1, sem1)

        def win2(i, _):
            w = i * 2
            gwait(buf0, sem0)
            scat_add(w, buf0, ssem0)
            gwait(buf1, sem1)
            scat_add(w + 1, buf1, ssem1)

            @pl.when(w + 2 < IWIN)
            def _():
                swait(buf0, ssem0)
                gather(w + 2, buf0, sem0)
                swait(buf1, ssem1)
                gather(w + 3, buf1, sem1)

            return 0

        lax.fori_loop(0, IWIN // 2, win2, 0)
        swait(buf0, ssem0)
        swait(buf1, ssem1)
        return 0

    lax.fori_loop(0, EPT // ICH, chunk, 0)
    plsc.subcore_barrier()
    pltpu.sync_copy(acc.at[pl.ds(r0, ROWS_PER_TILE)],
                    scat_ref.at[pl.ds(grow0, ROWS_PER_TILE)])


def _t1a_body(x_ref, nu_ref, wga_ref, bga_ref, wgb_ref,
              bgb_ref, w1_ref, b1_ref, u_ref):
    nu = nu_ref[0, 0]
    t = jnp.maximum(nu * wga_ref[...] + bga_ref[...], 0.0)
    logits = jnp.dot(t, wgb_ref[...], preferred_element_type=jnp.float32)
    logits = logits + bgb_ref[...]
    m = jnp.max(logits, axis=-1, keepdims=True)
    ex = jnp.exp(logits - m)
    fw = ex / jnp.sum(ex, axis=-1, keepdims=True)
    h0 = x_ref[...] * fw
    u = jnp.dot(h0, w1_ref[...], preferred_element_type=jnp.float32)
    u_ref[...] = u + b1_ref[...]


def _t1b_body(u_ref, d0_ref, d1_ref, g_ref, dis_ref):
    deg = d0_ref[...] + d1_ref[...] + 1.0
    dis = lax.rsqrt(jnp.maximum(deg, 1.0))
    dis_ref[...] = dis
    g = u_ref[...] * dis
    g_ref[0] = g[:, :HH]
    g_ref[1] = g[:, HH:]


def _t23_body(s_ref, dis_ref, w_ref, b_ref, g_ref):
    dis = dis_ref[...]
    s_full = jnp.concatenate([s_ref[0], s_ref[1]], axis=1)
    h = jnp.maximum(s_full * dis, 0.0)
    u = jnp.dot(h, w_ref[...], preferred_element_type=jnp.float32)
    u = u + b_ref[...]
    g = u * dis
    g_ref[0] = g[:, :HH]
    g_ref[1] = g[:, HH:]


def _t4_body(s_ref, dis_ref, nu_ref, wa1h_ref, wa1nu_ref, ba1_ref, wa2_ref,
             ba2_ref, wo1_ref, bo1_ref, wo2_ref, bo2_ref, out_ref):
    dis = dis_ref[...]
    h3 = jnp.concatenate([s_ref[0], s_ref[1]], axis=1) * dis
    nu = nu_ref[0, 0]
    a = jnp.dot(h3, wa1h_ref[...], preferred_element_type=jnp.float32)
    a = jnp.maximum(a + nu * wa1nu_ref[...] + ba1_ref[...], 0.0)
    att_l = jnp.dot(a, wa2_ref[...], preferred_element_type=jnp.float32)
    att = jax.nn.sigmoid(att_l + ba2_ref[...])
    ah = h3 * att
    z = jnp.dot(ah, wo1_ref[...], preferred_element_type=jnp.float32)
    z = jnp.maximum(z + bo1_ref[...], 0.0)
    o = jnp.dot(z, wo2_ref[...], preferred_element_type=jnp.float32)
    out_ref[...] = jax.nn.sigmoid(o + bo2_ref[...])


def _full(shape):
    return pl.BlockSpec(shape, lambda i: tuple(0 for _ in shape))


_t1a = pl.pallas_call(
    _t1a_body,
    grid=(GB,),
    in_specs=[
        pl.BlockSpec((R, D), lambda i: (i, 0)),
        _full((1, 1)),
        _full((1, 32)),
        _full((1, 32)),
        _full((32, D)),
        _full((1, D)),
        _full((D, H)),
        _full((1, H)),
    ],
    out_specs=pl.BlockSpec((R, H), lambda i: (i, 0)),
    out_shape=jax.ShapeDtypeStruct((NP, H), jnp.float32),
)

_t1b = pl.pallas_call(
    _t1b_body,
    grid=(GB,),
    in_specs=[
        pl.BlockSpec((R, H), lambda i: (i, 0)),
        pl.BlockSpec((R, 1), lambda i: (i, 0)),
        pl.BlockSpec((R, 1), lambda i: (i, 0)),
    ],
    out_specs=[
        pl.BlockSpec((2, R, HH), lambda i: (0, i, 0)),
        pl.BlockSpec((R, 1), lambda i: (i, 0)),
    ],
    out_shape=[
        jax.ShapeDtypeStruct((2, NP, HH), jnp.float32),
        jax.ShapeDtypeStruct((NP, 1), jnp.float32),
    ],
)

_t23 = pl.pallas_call(
    _t23_body,
    grid=(GB,),
    in_specs=[
        pl.BlockSpec((2, R, HH), lambda i: (0, i, 0)),
        pl.BlockSpec((R, 1), lambda i: (i, 0)),
        _full((H, H)),
        _full((1, H)),
    ],
    out_specs=pl.BlockSpec((2, R, HH), lambda i: (0, i, 0)),
    out_shape=jax.ShapeDtypeStruct((2, NP, HH), jnp.float32),
)

_t4 = pl.pallas_call(
    _t4_body,
    grid=(GB,),
    in_specs=[
        pl.BlockSpec((2, R, HH), lambda i: (0, i, 0)),
        pl.BlockSpec((R, 1), lambda i: (i, 0)),
        _full((1, 1)),
        _full((H, HH)),
        _full((1, HH)),
        _full((1, HH)),
        _full((HH, 1)),
        _full((1, 1)),
        _full((H, HH)),
        _full((1, HH)),
        _full((HH, 1)),
        _full((1, 1)),
    ],
    out_specs=pl.BlockSpec((R, 1), lambda i: (i, 0)),
    out_shape=jax.ShapeDtypeStruct((NP, 1), jnp.float32),
)


def kernel(x, edge_index, nu, Wg_a, bg_a, Wg_b, bg_b, W1, b1, W2, b2, W3, b3,
           Wa1, ba1, Wa2, ba2, Wo1, bo1, Wo2, bo2):
    src = edge_index[0].astype(jnp.int32)
    dst = edge_index[1].astype(jnp.int32)
    src3 = jnp.concatenate([src, src + NP]).reshape(2, NTILE, E // NTILE)
    dst3 = jnp.concatenate([dst, dst]).reshape(2, NTILE, E // NTILE)
    pad_s = jnp.broadcast_to(jnp.arange(CH_PAD, dtype=jnp.int32) % N,
                             (2, NTILE, CH_PAD))
    pad_d = jnp.broadcast_to(
        N + jnp.arange(CH_PAD, dtype=jnp.int32) % (NP - N),
        (2, NTILE, CH_PAD))
    src2 = jnp.concatenate([src3, pad_s], axis=2).reshape(-1)
    dst2 = jnp.concatenate([dst3, pad_d], axis=2).reshape(-1)
    xp = jnp.pad(x, ((0, NP - N), (0, 0)))

    part = _deg_sc(dst)
    d0 = part[0:NP].reshape(NP, 1)
    d1 = part[NP:2 * NP].reshape(NP, 1)

    nu2 = nu.reshape(1, 1)
    u1 = _t1a(xp, nu2, Wg_a, bg_a.reshape(1, 32), Wg_b,
              bg_b.reshape(1, D), W1, b1.reshape(1, H))
    g, dis = _t1b(u1, d0, d1)

    s = _conv_sc(g.reshape(2 * NP, HH), src2, dst2).reshape(2, NP, HH)
    g = _t23(s, dis, W2, b2.reshape(1, H))
    s = _conv_sc(g.reshape(2 * NP, HH), src2, dst2).reshape(2, NP, HH)
    g = _t23(s, dis, W3, b3.reshape(1, H))
    s = _conv_sc(g.reshape(2 * NP, HH), src2, dst2).reshape(2, NP, HH)

    out = _t4(s, dis, nu2, Wa1[:H], Wa1[H:H + 1], ba1.reshape(1, HH),
              Wa2, ba2.reshape(1, 1), Wo1, bo1.reshape(1, HH),
              Wo2, bo2.reshape(1, 1))
    return out[:N]


# speedup vs baseline: 15.4672x; 1.0761x over previous
import functools

import jax
import jax.numpy as jnp
from jax import lax
from jax.experimental import pallas as pl
from jax.experimental.pallas import tpu as pltpu
from jax.experimental.pallas import tpu_sc as plsc

N = 10000
NP = 10240
E = 320000
D = 128
H = 256
HH = H // 2

NTILE = 16
ROWS_PER_TILE = NP // NTILE
CH = 128
WPT = 160
EPT = CH * WPT
CH_PAD = EPT - E // NTILE
ICH = 5120
IWIN = ICH // CH

DEG_PER_TILE = NP // NTILE
DEG_EDGES_PER_TILE = E // (2 * NTILE)
DEG_NWIN = DEG_EDGES_PER_TILE // CH
DEG_TAIL = DEG_EDGES_PER_TILE - DEG_NWIN * CH

R = 1280
GB = NP // R

_sc_mesh = plsc.VectorSubcoreMesh(core_axis_name="c", subcore_axis_name="s")


@functools.partial(
    pl.kernel,
    out_type=jax.ShapeDtypeStruct((2 * NP,), jnp.float32),
    mesh=_sc_mesh,
    scratch_types=[
        pltpu.VMEM_SHARED((NP,), jnp.float32),
        pltpu.VMEM((DEG_EDGES_PER_TILE,), jnp.int32),
        pltpu.VMEM((CH,), jnp.float32),
        pltpu.VMEM((DEG_PER_TILE,), jnp.float32),
        pltpu.SemaphoreType.DMA,
    ],
)
def _deg_sc(dst_ref, part_ref, acc, idxb, ones_b, zbuf, ssem):
    c = lax.axis_index("c")
    sid = lax.axis_index("s")

    def fill_z(i, _):
        zbuf[pl.ds(i * 16, 16)] = jnp.zeros((16,), jnp.float32)
        return 0

    lax.fori_loop(0, DEG_PER_TILE // 16, fill_z, 0)

    def fill_o(i, _):
        ones_b[pl.ds(i * 16, 16)] = jnp.ones((16,), jnp.float32)
        return 0

    lax.fori_loop(0, CH // 16, fill_o, 0)

    base = pl.multiple_of(c * (E // 2) + sid * DEG_EDGES_PER_TILE, 8)
    pltpu.sync_copy(dst_ref.at[pl.ds(base, DEG_EDGES_PER_TILE)], idxb)

    my0 = pl.multiple_of(sid * DEG_PER_TILE, 8)
    pltpu.sync_copy(zbuf, acc.at[pl.ds(my0, DEG_PER_TILE)])
    plsc.subcore_barrier()

    def win(w, _):
        idx = idxb.at[pl.ds(pl.multiple_of(w * CH, 8), CH)]
        pltpu.async_copy(ones_b, acc.at[idx], ssem, add=True)
        return 0

    lax.fori_loop(0, DEG_NWIN, win, 0)
    tidx = idxb.at[pl.ds(pl.multiple_of(DEG_NWIN * CH, 8), DEG_TAIL)]
    pltpu.async_copy(ones_b.at[pl.ds(0, DEG_TAIL)], acc.at[tidx], ssem,
                     add=True)

    def drain(w, _):
        idx = idxb.at[pl.ds(0, CH)]
        pltpu.make_async_copy(ones_b, acc.at[idx], ssem).wait()
        return 0

    lax.fori_loop(0, DEG_NWIN, drain, 0)
    pltpu.make_async_copy(ones_b.at[pl.ds(0, DEG_TAIL)],
                          acc.at[idxb.at[pl.ds(0, DEG_TAIL)]], ssem).wait()
    plsc.subcore_barrier()
    out0 = pl.multiple_of(c * NP + sid * DEG_PER_TILE, 8)
    pltpu.sync_copy(acc.at[pl.ds(my0, DEG_PER_TILE)],
                    part_ref.at[pl.ds(out0, DEG_PER_TILE)])


@functools.partial(
    pl.kernel,
    out_type=jax.ShapeDtypeStruct((2 * NP, HH), jnp.float32),
    mesh=_sc_mesh,
    scratch_types=[
        pltpu.VMEM_SHARED((NP, HH), jnp.float32),
        pltpu.VMEM((ICH,), jnp.int32),
        pltpu.VMEM((ICH,), jnp.int32),
        pltpu.VMEM((CH, HH), jnp.float32),
        pltpu.VMEM((CH, HH), jnp.float32),
        pltpu.SemaphoreType.DMA,
        pltpu.SemaphoreType.DMA,
        pltpu.SemaphoreType.DMA,
        pltpu.SemaphoreType.DMA,
    ],
)
def _conv_sc(gcat_ref, src2_ref, dst2_ref, scat_ref, acc, sbuf, dbuf,
             buf0, buf1, sem0, sem1, ssem0, ssem1):
    c = lax.axis_index("c")
    sid = lax.axis_index("s")

    r0 = pl.multiple_of(sid * ROWS_PER_TILE, 8)
    grow0 = pl.multiple_of(c * NP + sid * ROWS_PER_TILE, 8)
    ebase = pl.multiple_of((c * NTILE + sid) * EPT, 8)

    pltpu.sync_copy(gcat_ref.at[pl.ds(grow0, ROWS_PER_TILE)],
                    acc.at[pl.ds(r0, ROWS_PER_TILE)])
    plsc.subcore_barrier()

    def gather(w, buf, sem):
        idx = sbuf.at[pl.ds(pl.multiple_of(w * CH, 8), CH)]
        pltpu.async_copy(gcat_ref.at[idx], buf, sem)

    def scat_add(w, buf, sem):
        idx = dbuf.at[pl.ds(pl.multiple_of(w * CH, 8), CH)]
        pltpu.async_copy(buf, acc.at[idx], sem, add=True)

    def gwait(buf, sem):
        pltpu.make_async_copy(gcat_ref.at[sbuf.at[pl.ds(0, CH)]],
                              buf, sem).wait()

    def swait(buf, sem):
        pltpu.make_async_copy(buf, acc.at[dbuf.at[pl.ds(0, CH)]],
                              sem).wait()

    def chunk(k, _):
        eoff = pl.multiple_of(ebase + k * ICH, 8)
        pltpu.sync_copy(src2_ref.at[pl.ds(eoff, ICH)], sbuf)
        pltpu.sync_copy(dst2_ref.at[pl.ds(eoff, ICH)], dbuf)
        gather(0, buf0, sem0)
        gather(1, buf1, sem1)

        def win2(i, _):
            w = i * 2
            gwait(buf0, sem0)
            scat_add(w, buf0, ssem0)
            gwait(buf1, sem1)
            scat_add(w + 1, buf1, ssem1)

            @pl.when(w + 2 < IWIN)
            def _():
                swait(buf0, ssem0)
                gather(w + 2, buf0, sem0)
                swait(buf1, ssem1)
                gather(w + 3, buf1, sem1)

            return 0

        lax.fori_loop(0, IWIN // 2, win2, 0)
        swait(buf0, ssem0)
        swait(buf1, ssem1)
        return 0

    lax.fori_loop(0, EPT // ICH, chunk, 0)
    plsc.subcore_barrier()
    pltpu.sync_copy(acc.at[pl.ds(r0, ROWS_PER_TILE)],
                    scat_ref.at[pl.ds(grow0, ROWS_PER_TILE)])


def _t1a_body(x_ref, nu_ref, wga_ref, bga_ref, wgb_ref,
              bgb_ref, w1_ref, b1_ref, u_ref):
    nu = nu_ref[0, 0]
    t = jnp.maximum(nu * wga_ref[...] + bga_ref[...], 0.0)
    logits = jnp.dot(t, wgb_ref[...], preferred_element_type=jnp.float32)
    logits = logits + bgb_ref[...]
    m = jnp.max(logits, axis=-1, keepdims=True)
    ex = jnp.exp(logits - m)
    fw = ex / jnp.sum(ex, axis=-1, keepdims=True)
    h0 = x_ref[...] * fw
    u = jnp.dot(h0, w1_ref[...], preferred_element_type=jnp.float32)
    u_ref[...] = u + b1_ref[...]


def _t1b_body(u_ref, d0_ref, d1_ref, g_ref, dis_ref):
    deg = d0_ref[...] + d1_ref[...] + 1.0
    dis = lax.rsqrt(jnp.maximum(deg, 1.0))
    dis_ref[...] = dis
    g = u_ref[...] * dis
    g_ref[0] = g[:, :HH]
    g_ref[1] = g[:, HH:]


def _t23_body(s_ref, dis_ref, w_ref, b_ref, g_ref):
    dis = dis_ref[...]
    s_full = jnp.concatenate([s_ref[0], s_ref[1]], axis=1)
    h = jnp.maximum(s_full * dis, 0.0)
    u = jnp.dot(h, w_ref[...], preferred_element_type=jnp.float32)
    u = u + b_ref[...]
    g = u * dis
    g_ref[0] = g[:, :HH]
    g_ref[1] = g[:, HH:]


def _t4_body(s_ref, dis_ref, nu_ref, wa1h_ref, wa1nu_ref, ba1_ref, wa2_ref,
             ba2_ref, wo1_ref, bo1_ref, wo2_ref, bo2_ref, out_ref):
    dis = dis_ref[...]
    h3 = jnp.concatenate([s_ref[0], s_ref[1]], axis=1) * dis
    nu = nu_ref[0, 0]
    a = jnp.dot(h3, wa1h_ref[...], preferred_element_type=jnp.float32)
    a = jnp.maximum(a + nu * wa1nu_ref[...] + ba1_ref[...], 0.0)
    att_l = jnp.dot(a, wa2_ref[...], preferred_element_type=jnp.float32)
    att = jax.nn.sigmoid(att_l + ba2_ref[...])
    ah = h3 * att
    z = jnp.dot(ah, wo1_ref[...], preferred_element_type=jnp.float32)
    z = jnp.maximum(z + bo1_ref[...], 0.0)
    o = jnp.dot(z, wo2_ref[...], preferred_element_type=jnp.float32)
    out_ref[...] = jax.nn.sigmoid(o + bo2_ref[...])


def _full(shape):
    return pl.BlockSpec(shape, lambda i: tuple(0 for _ in shape))


_t1a = pl.pallas_call(
    _t1a_body,
    grid=(GB,),
    in_specs=[
        pl.BlockSpec((R, D), lambda i: (i, 0)),
        _full((1, 1)),
        _full((1, 32)),
        _full((1, 32)),
        _full((32, D)),
        _full((1, D)),
        _full((D, H)),
        _full((1, H)),
    ],
    out_specs=pl.BlockSpec((R, H), lambda i: (i, 0)),
    out_shape=jax.ShapeDtypeStruct((NP, H), jnp.float32),
)

_t1b = pl.pallas_call(
    _t1b_body,
    grid=(GB,),
    in_specs=[
        pl.BlockSpec((R, H), lambda i: (i, 0)),
        pl.BlockSpec((R, 1), lambda i: (i, 0)),
        pl.BlockSpec((R, 1), lambda i: (i, 0)),
    ],
    out_specs=[
        pl.BlockSpec((2, R, HH), lambda i: (0, i, 0)),
        pl.BlockSpec((R, 1), lambda i: (i, 0)),
    ],
    out_shape=[
        jax.ShapeDtypeStruct((2, NP, HH), jnp.float32),
        jax.ShapeDtypeStruct((NP, 1), jnp.float32),
    ],
)

_t23 = pl.pallas_call(
    _t23_body,
    grid=(GB,),
    in_specs=[
        pl.BlockSpec((2, R, HH), lambda i: (0, i, 0)),
        pl.BlockSpec((R, 1), lambda i: (i, 0)),
        _full((H, H)),
        _full((1, H)),
    ],
    out_specs=pl.BlockSpec((2, R, HH), lambda i: (0, i, 0)),
    out_shape=jax.ShapeDtypeStruct((2, NP, HH), jnp.float32),
)

_t4 = pl.pallas_call(
    _t4_body,
    grid=(GB,),
    in_specs=[
        pl.BlockSpec((2, R, HH), lambda i: (0, i, 0)),
        pl.BlockSpec((R, 1), lambda i: (i, 0)),
        _full((1, 1)),
        _full((H, HH)),
        _full((1, HH)),
        _full((1, HH)),
        _full((HH, 1)),
        _full((1, 1)),
        _full((H, HH)),
        _full((1, HH)),
        _full((HH, 1)),
        _full((1, 1)),
    ],
    out_specs=pl.BlockSpec((R, 1), lambda i: (i, 0)),
    out_shape=jax.ShapeDtypeStruct((NP, 1), jnp.float32),
)


def kernel(x, edge_index, nu, Wg_a, bg_a, Wg_b, bg_b, W1, b1, W2, b2, W3, b3,
           Wa1, ba1, Wa2, ba2, Wo1, bo1, Wo2, bo2):
    src = edge_index[0].astype(jnp.int32)
    dst = edge_index[1].astype(jnp.int32)
    src3 = jnp.concatenate([src, src + NP]).reshape(2, NTILE, E // NTILE)
    dst3 = jnp.concatenate([dst, dst]).reshape(2, NTILE, E // NTILE)
    pad_s = jnp.broadcast_to(jnp.arange(CH_PAD, dtype=jnp.int32) % N,
                             (2, NTILE, CH_PAD))
    pad_d = jnp.broadcast_to(
        N + jnp.arange(CH_PAD, dtype=jnp.int32) % (NP - N),
        (2, NTILE, CH_PAD))
    src2 = jnp.concatenate([src3, pad_s], axis=2).reshape(-1)
    dst2 = jnp.concatenate([dst3, pad_d], axis=2).reshape(-1)
    xp = jnp.pad(x, ((0, NP - N), (0, 0)))

    part = _deg_sc(dst)
    d0 = part[0:NP].reshape(NP, 1)
    d1 = part[NP:2 * NP].reshape(NP, 1)

    nu2 = nu.reshape(1, 1)
    u1 = _t1a(xp, nu2, Wg_a, bg_a.reshape(1, 32), Wg_b,
              bg_b.reshape(1, D), W1, b1.reshape(1, H))
    g, dis = _t1b(u1, d0, d1)

    s = _conv_sc(g.reshape(2 * NP, HH), src2, dst2).reshape(2, NP, HH)
    g = _t23(s, dis, W2, b2.reshape(1, H))
    s = _conv_sc(g.reshape(2 * NP, HH), src2, dst2).reshape(2, NP, HH)
    g = _t23(s, dis, W3, b3.reshape(1, H))
    s = _conv_sc(g.reshape(2 * NP, HH), src2, dst2).reshape(2, NP, HH)

    out = _t4(s, dis, nu2, Wa1[:H], Wa1[H:H + 1], ba1.reshape(1, HH),
              Wa2, ba2.reshape(1, 1), Wo1, bo1.reshape(1, HH),
              Wo2, bo2.reshape(1, 1))
    return out[:N]


# speedup vs baseline: 15.6312x; 1.0106x over previous
import functools

import jax
import jax.numpy as jnp
from jax import lax
from jax.experimental import pallas as pl
from jax.experimental.pallas import tpu as pltpu
from jax.experimental.pallas import tpu_sc as plsc

N = 10000
NP = 10240
E = 320000
D = 128
H = 256
HH = H // 2

NTILE = 16
ROWS_PER_TILE = NP // NTILE
CH = 128
WPT = 160
EPT = CH * WPT
CH_PAD = EPT - E // NTILE
ICH = 5120
IWIN = ICH // CH

DEG_PER_TILE = NP // NTILE
DEG_EDGES_PER_TILE = EPT // 2
DEG_NWIN = DEG_EDGES_PER_TILE // CH

R = 1280
GB = NP // R

_sc_mesh = plsc.VectorSubcoreMesh(core_axis_name="c", subcore_axis_name="s")


@functools.partial(
    pl.kernel,
    out_type=jax.ShapeDtypeStruct((2 * NP,), jnp.float32),
    mesh=_sc_mesh,
    scratch_types=[
        pltpu.VMEM_SHARED((NP,), jnp.float32),
        pltpu.VMEM((DEG_EDGES_PER_TILE,), jnp.int32),
        pltpu.VMEM((CH,), jnp.float32),
        pltpu.VMEM((DEG_PER_TILE,), jnp.float32),
        pltpu.SemaphoreType.DMA,
    ],
)
def _deg_sc(dst_ref, part_ref, acc, idxb, ones_b, zbuf, ssem):
    c = lax.axis_index("c")
    sid = lax.axis_index("s")

    def fill_z(i, _):
        zbuf[pl.ds(i * 16, 16)] = jnp.zeros((16,), jnp.float32)
        return 0

    lax.fori_loop(0, DEG_PER_TILE // 16, fill_z, 0)

    def fill_o(i, _):
        ones_b[pl.ds(i * 16, 16)] = jnp.ones((16,), jnp.float32)
        return 0

    lax.fori_loop(0, CH // 16, fill_o, 0)

    base = pl.multiple_of(sid * EPT + c * DEG_EDGES_PER_TILE, 8)
    pltpu.sync_copy(dst_ref.at[pl.ds(base, DEG_EDGES_PER_TILE)], idxb)

    my0 = pl.multiple_of(sid * DEG_PER_TILE, 8)
    pltpu.sync_copy(zbuf, acc.at[pl.ds(my0, DEG_PER_TILE)])
    plsc.subcore_barrier()

    def win(w, _):
        idx = idxb.at[pl.ds(pl.multiple_of(w * CH, 8), CH)]
        pltpu.async_copy(ones_b, acc.at[idx], ssem, add=True)
        return 0

    lax.fori_loop(0, DEG_NWIN, win, 0)

    def drain(w, _):
        idx = idxb.at[pl.ds(0, CH)]
        pltpu.make_async_copy(ones_b, acc.at[idx], ssem).wait()
        return 0

    lax.fori_loop(0, DEG_NWIN, drain, 0)
    plsc.subcore_barrier()
    out0 = pl.multiple_of(c * NP + sid * DEG_PER_TILE, 8)
    pltpu.sync_copy(acc.at[pl.ds(my0, DEG_PER_TILE)],
                    part_ref.at[pl.ds(out0, DEG_PER_TILE)])


@functools.partial(
    pl.kernel,
    out_type=jax.ShapeDtypeStruct((2 * NP, HH), jnp.float32),
    mesh=_sc_mesh,
    scratch_types=[
        pltpu.VMEM_SHARED((NP, HH), jnp.float32),
        pltpu.VMEM((ICH,), jnp.int32),
        pltpu.VMEM((ICH,), jnp.int32),
        pltpu.VMEM((CH, HH), jnp.float32),
        pltpu.VMEM((CH, HH), jnp.float32),
        pltpu.SemaphoreType.DMA,
        pltpu.SemaphoreType.DMA,
        pltpu.SemaphoreType.DMA,
        pltpu.SemaphoreType.DMA,
    ],
)
def _conv_sc(gcat_ref, src2_ref, dst2_ref, scat_ref, acc, sbuf, dbuf,
             buf0, buf1, sem0, sem1, ssem0, ssem1):
    c = lax.axis_index("c")
    sid = lax.axis_index("s")

    r0 = pl.multiple_of(sid * ROWS_PER_TILE, 8)
    grow0 = pl.multiple_of(c * NP + sid * ROWS_PER_TILE, 8)
    ebase = pl.multiple_of((c * NTILE + sid) * EPT, 8)

    pltpu.sync_copy(gcat_ref.at[pl.ds(grow0, ROWS_PER_TILE)],
                    acc.at[pl.ds(r0, ROWS_PER_TILE)])
    plsc.subcore_barrier()

    def gather(w, buf, sem):
        idx = sbuf.at[pl.ds(pl.multiple_of(w * CH, 8), CH)]
        pltpu.async_copy(gcat_ref.at[idx], buf, sem)

    def scat_add(w, buf, sem):
        idx = dbuf.at[pl.ds(pl.multiple_of(w * CH, 8), CH)]
        pltpu.async_copy(buf, acc.at[idx], sem, add=True)

    def gwait(buf, sem):
        pltpu.make_async_copy(gcat_ref.at[sbuf.at[pl.ds(0, CH)]],
                              buf, sem).wait()

    def swait(buf, sem):
        pltpu.make_async_copy(buf, acc.at[dbuf.at[pl.ds(0, CH)]],
                              sem).wait()

    def chunk(k, _):
        eoff = pl.multiple_of(ebase + k * ICH, 8)
        pltpu.sync_copy(src2_ref.at[pl.ds(eoff, ICH)], sbuf)
        pltpu.sync_copy(dst2_ref.at[pl.ds(eoff, ICH)], dbuf)
        gather(0, buf0, sem0)
        gather(1, buf1, sem1)

        def win2(i, _):
            w = i * 2
            gwait(buf0, sem0)
            scat_add(w, buf0, ssem0)
            gwait(buf1, sem1)
            scat_add(w + 1, buf1, ssem1)

            @pl.when(w + 2 < IWIN)
            def _():
                swait(buf0, ssem0)
                gather(w + 2, buf0, sem0)
                swait(buf1, ssem1)
                gather(w + 3, buf1, sem1)

            return 0

        lax.fori_loop(0, IWIN // 2, win2, 0)
        swait(buf0, ssem0)
        swait(buf1, ssem1)
        return 0

    lax.fori_loop(0, EPT // ICH, chunk, 0)
    plsc.subcore_barrier()
    pltpu.sync_copy(acc.at[pl.ds(r0, ROWS_PER_TILE)],
                    scat_ref.at[pl.ds(grow0, ROWS_PER_TILE)])


def _t1a_body(x_ref, nu_ref, wga_ref, bga_ref, wgb_ref,
              bgb_ref, w1_ref, b1_ref, u_ref):
    nu = nu_ref[0, 0]
    t = jnp.maximum(nu * wga_ref[...] + bga_ref[...], 0.0)
    logits = jnp.dot(t, wgb_ref[...], preferred_element_type=jnp.float32)
    logits = logits + bgb_ref[...]
    m = jnp.max(logits, axis=-1, keepdims=True)
    ex = jnp.exp(logits - m)
    fw = ex / jnp.sum(ex, axis=-1, keepdims=True)
    h0 = x_ref[...] * fw
    u = jnp.dot(h0, w1_ref[...], preferred_element_type=jnp.float32)
    u_ref[...] = u + b1_ref[...]


def _t1b_body(u_ref, d0_ref, d1_ref, g_ref, dis_ref):
    deg = d0_ref[...] + d1_ref[...] + 1.0
    dis = lax.rsqrt(jnp.maximum(deg, 1.0))
    dis_ref[...] = dis
    g = u_ref[...] * dis
    g_ref[0] = g[:, :HH]
    g_ref[1] = g[:, HH:]


def _t23_body(s_ref, dis_ref, w_ref, b_ref, g_ref):
    dis = dis_ref[...]
    s_full = jnp.concatenate([s_ref[0], s_ref[1]], axis=1)
    h = jnp.maximum(s_full * dis, 0.0)
    u = jnp.dot(h, w_ref[...], preferred_element_type=jnp.float32)
    u = u + b_ref[...]
    g = u * dis
    g_ref[0] = g[:, :HH]
    g_ref[1] = g[:, HH:]


def _t4_body(s_ref, dis_ref, nu_ref, wa1h_ref, wa1nu_ref, ba1_ref, wa2_ref,
             ba2_ref, wo1_ref, bo1_ref, wo2_ref, bo2_ref, out_ref):
    dis = dis_ref[...]
    h3 = jnp.concatenate([s_ref[0], s_ref[1]], axis=1) * dis
    nu = nu_ref[0, 0]
    a = jnp.dot(h3, wa1h_ref[...], preferred_element_type=jnp.float32)
    a = jnp.maximum(a + nu * wa1nu_ref[...] + ba1_ref[...], 0.0)
    att_l = jnp.dot(a, wa2_ref[...], preferred_element_type=jnp.float32)
    att = jax.nn.sigmoid(att_l + ba2_ref[...])
    ah = h3 * att
    z = jnp.dot(ah, wo1_ref[...], preferred_element_type=jnp.float32)
    z = jnp.maximum(z + bo1_ref[...], 0.0)
    o = jnp.dot(z, wo2_ref[...], preferred_element_type=jnp.float32)
    out_ref[...] = jax.nn.sigmoid(o + bo2_ref[...])


def _full(shape):
    return pl.BlockSpec(shape, lambda i: tuple(0 for _ in shape))


_t1a = pl.pallas_call(
    _t1a_body,
    grid=(GB,),
    in_specs=[
        pl.BlockSpec((R, D), lambda i: (i, 0)),
        _full((1, 1)),
        _full((1, 32)),
        _full((1, 32)),
        _full((32, D)),
        _full((1, D)),
        _full((D, H)),
        _full((1, H)),
    ],
    out_specs=pl.BlockSpec((R, H), lambda i: (i, 0)),
    out_shape=jax.ShapeDtypeStruct((NP, H), jnp.float32),
)

_t1b = pl.pallas_call(
    _t1b_body,
    grid=(GB,),
    in_specs=[
        pl.BlockSpec((R, H), lambda i: (i, 0)),
        pl.BlockSpec((R, 1), lambda i: (i, 0)),
        pl.BlockSpec((R, 1), lambda i: (i, 0)),
    ],
    out_specs=[
        pl.BlockSpec((2, R, HH), lambda i: (0, i, 0)),
        pl.BlockSpec((R, 1), lambda i: (i, 0)),
    ],
    out_shape=[
        jax.ShapeDtypeStruct((2, NP, HH), jnp.float32),
        jax.ShapeDtypeStruct((NP, 1), jnp.float32),
    ],
)

_t23 = pl.pallas_call(
    _t23_body,
    grid=(GB,),
    in_specs=[
        pl.BlockSpec((2, R, HH), lambda i: (0, i, 0)),
        pl.BlockSpec((R, 1), lambda i: (i, 0)),
        _full((H, H)),
        _full((1, H)),
    ],
    out_specs=pl.BlockSpec((2, R, HH), lambda i: (0, i, 0)),
    out_shape=jax.ShapeDtypeStruct((2, NP, HH), jnp.float32),
)

_t4 = pl.pallas_call(
    _t4_body,
    grid=(GB,),
    in_specs=[
        pl.BlockSpec((2, R, HH), lambda i: (0, i, 0)),
        pl.BlockSpec((R, 1), lambda i: (i, 0)),
        _full((1, 1)),
        _full((H, HH)),
        _full((1, HH)),
        _full((1, HH)),
        _full((HH, 1)),
        _full((1, 1)),
        _full((H, HH)),
        _full((1, HH)),
        _full((HH, 1)),
        _full((1, 1)),
    ],
    out_specs=pl.BlockSpec((R, 1), lambda i: (i, 0)),
    out_shape=jax.ShapeDtypeStruct((NP, 1), jnp.float32),
)


def _eb_body(ei_ref, ps_ref, pd_ref, src2_ref, dst2_ref):
    i = pl.program_id(0)
    off = jnp.where(i >= 4, NP, 0).astype(jnp.int32)
    ept0 = E // NTILE
    for t in range(4):
        src2_ref[pl.ds(t * EPT, ept0)] = ei_ref[0, pl.ds(t * ept0, ept0)] + off
        dst2_ref[pl.ds(t * EPT, ept0)] = ei_ref[1, pl.ds(t * ept0, ept0)]
        src2_ref[pl.ds(t * EPT + ept0, CH_PAD)] = ps_ref[...] + off
        dst2_ref[pl.ds(t * EPT + ept0, CH_PAD)] = pd_ref[...]


_edge_build = pl.pallas_call(
    _eb_body,
    grid=(8,),
    in_specs=[
        pl.BlockSpec((2, 4 * (E // NTILE)), lambda i: (0, i % 4)),
        pl.BlockSpec((CH_PAD,), lambda i: (0,)),
        pl.BlockSpec((CH_PAD,), lambda i: (0,)),
    ],
    out_specs=[
        pl.BlockSpec((4 * EPT,), lambda i: (i,)),
        pl.BlockSpec((4 * EPT,), lambda i: (i,)),
    ],
    out_shape=[
        jax.ShapeDtypeStruct((2 * NTILE * EPT,), jnp.int32),
        jax.ShapeDtypeStruct((2 * NTILE * EPT,), jnp.int32),
    ],
)


def kernel(x, edge_index, nu, Wg_a, bg_a, Wg_b, bg_b, W1, b1, W2, b2, W3, b3,
           Wa1, ba1, Wa2, ba2, Wo1, bo1, Wo2, bo2):
    ei = edge_index.astype(jnp.int32)
    pad_s = jnp.arange(CH_PAD, dtype=jnp.int32) % N
    pad_d = N + jnp.arange(CH_PAD, dtype=jnp.int32) % (NP - N)
    src2, dst2 = _edge_build(ei, pad_s, pad_d)
    xp = jnp.pad(x, ((0, NP - N), (0, 0)))

    part = _deg_sc(dst2)
    d0 = part[0:NP].reshape(NP, 1)
    d1 = part[NP:2 * NP].reshape(NP, 1)

    nu2 = nu.reshape(1, 1)
    u1 = _t1a(xp, nu2, Wg_a, bg_a.reshape(1, 32), Wg_b,
              bg_b.reshape(1, D), W1, b1.reshape(1, H))
    g, dis = _t1b(u1, d0, d1)

    s = _conv_sc(g.reshape(2 * NP, HH), src2, dst2).reshape(2, NP, HH)
    g = _t23(s, dis, W2, b2.reshape(1, H))
    s = _conv_sc(g.reshape(2 * NP, HH), src2, dst2).reshape(2, NP, HH)
    g = _t23(s, dis, W3, b3.reshape(1, H))
    s = _conv_sc(g.reshape(2 * NP, HH), src2, dst2).reshape(2, NP, HH)

    out = _t4(s, dis, nu2, Wa1[:H], Wa1[H:H + 1], ba1.reshape(1, HH),
              Wa2, ba2.reshape(1, 1), Wo1, bo1.reshape(1, HH),
              Wo2, bo2.reshape(1, 1))
    return out[:N]


# speedup vs baseline: 15.8686x; 1.0152x over previous
import functools

import jax
import jax.numpy as jnp
from jax import lax
from jax.experimental import pallas as pl
from jax.experimental.pallas import tpu as pltpu
from jax.experimental.pallas import tpu_sc as plsc

N = 10000
NP = 10240
E = 320000
D = 128
H = 256
HH = H // 2

NTILE = 16
ROWS_PER_TILE = NP // NTILE
CH = 128
WPT = 160
EPT = CH * WPT
CH_PAD = EPT - E // NTILE
ICH = 2560
IWIN = ICH // CH
NCHUNK = EPT // ICH

DEG_PER_TILE = NP // NTILE
DEG_EDGES_PER_TILE = EPT // 2
DEG_NWIN = DEG_EDGES_PER_TILE // CH

R = 1280
GB = NP // R

_sc_mesh = plsc.VectorSubcoreMesh(core_axis_name="c", subcore_axis_name="s")


@functools.partial(
    pl.kernel,
    out_type=jax.ShapeDtypeStruct((2 * NP,), jnp.float32),
    mesh=_sc_mesh,
    scratch_types=[
        pltpu.VMEM_SHARED((NP,), jnp.float32),
        pltpu.VMEM((DEG_EDGES_PER_TILE,), jnp.int32),
        pltpu.VMEM((CH,), jnp.float32),
        pltpu.VMEM((DEG_PER_TILE,), jnp.float32),
        pltpu.SemaphoreType.DMA,
    ],
)
def _deg_sc(dst_ref, part_ref, acc, idxb, ones_b, zbuf, ssem):
    c = lax.axis_index("c")
    sid = lax.axis_index("s")

    def fill_z(i, _):
        zbuf[pl.ds(i * 16, 16)] = jnp.zeros((16,), jnp.float32)
        return 0

    lax.fori_loop(0, DEG_PER_TILE // 16, fill_z, 0)

    def fill_o(i, _):
        ones_b[pl.ds(i * 16, 16)] = jnp.ones((16,), jnp.float32)
        return 0

    lax.fori_loop(0, CH // 16, fill_o, 0)

    base = pl.multiple_of(sid * EPT + c * DEG_EDGES_PER_TILE, 8)
    pltpu.sync_copy(dst_ref.at[pl.ds(base, DEG_EDGES_PER_TILE)], idxb)

    my0 = pl.multiple_of(sid * DEG_PER_TILE, 8)
    pltpu.sync_copy(zbuf, acc.at[pl.ds(my0, DEG_PER_TILE)])
    plsc.subcore_barrier()

    def win(w, _):
        idx = idxb.at[pl.ds(pl.multiple_of(w * CH, 8), CH)]
        pltpu.async_copy(ones_b, acc.at[idx], ssem, add=True)
        return 0

    lax.fori_loop(0, DEG_NWIN, win, 0)

    def drain(w, _):
        idx = idxb.at[pl.ds(0, CH)]
        pltpu.make_async_copy(ones_b, acc.at[idx], ssem).wait()
        return 0

    lax.fori_loop(0, DEG_NWIN, drain, 0)
    plsc.subcore_barrier()
    out0 = pl.multiple_of(c * NP + sid * DEG_PER_TILE, 8)
    pltpu.sync_copy(acc.at[pl.ds(my0, DEG_PER_TILE)],
                    part_ref.at[pl.ds(out0, DEG_PER_TILE)])


@functools.partial(
    pl.kernel,
    out_type=jax.ShapeDtypeStruct((2 * NP, HH), jnp.float32),
    mesh=_sc_mesh,
    scratch_types=[
        pltpu.VMEM_SHARED((NP, HH), jnp.float32),
        pltpu.VMEM((ICH,), jnp.int32),
        pltpu.VMEM((ICH,), jnp.int32),
        pltpu.VMEM((ICH,), jnp.int32),
        pltpu.VMEM((ICH,), jnp.int32),
        pltpu.VMEM((CH, HH), jnp.float32),
        pltpu.VMEM((CH, HH), jnp.float32),
        pltpu.SemaphoreType.DMA,
        pltpu.SemaphoreType.DMA,
        pltpu.SemaphoreType.DMA,
        pltpu.SemaphoreType.DMA,
        pltpu.SemaphoreType.DMA,
    ],
)
def _conv_sc(gcat_ref, src2_ref, dst2_ref, scat_ref, acc, sbuf0, dbuf0,
             sbuf1, dbuf1, buf0, buf1, sem0, sem1, ssem0, ssem1, isem):
    c = lax.axis_index("c")
    sid = lax.axis_index("s")

    r0 = pl.multiple_of(sid * ROWS_PER_TILE, 8)
    grow0 = pl.multiple_of(c * NP + sid * ROWS_PER_TILE, 8)
    ebase = pl.multiple_of((c * NTILE + sid) * EPT, 8)

    pltpu.sync_copy(gcat_ref.at[pl.ds(grow0, ROWS_PER_TILE)],
                    acc.at[pl.ds(r0, ROWS_PER_TILE)])
    plsc.subcore_barrier()

    def prefetch(k, sb, db):
        eoff = pl.multiple_of(ebase + k * ICH, 8)
        pltpu.async_copy(src2_ref.at[pl.ds(eoff, ICH)], sb, isem)
        pltpu.async_copy(dst2_ref.at[pl.ds(eoff, ICH)], db, isem)

    def iwait(sb, db):
        pltpu.make_async_copy(src2_ref.at[pl.ds(ebase, ICH)], sb, isem).wait()
        pltpu.make_async_copy(dst2_ref.at[pl.ds(ebase, ICH)], db, isem).wait()

    def run_windows(sbuf, dbuf):
        def gather(w, buf, sem):
            idx = sbuf.at[pl.ds(pl.multiple_of(w * CH, 8), CH)]
            pltpu.async_copy(gcat_ref.at[idx], buf, sem)

        def scat_add(w, buf, sem):
            idx = dbuf.at[pl.ds(pl.multiple_of(w * CH, 8), CH)]
            pltpu.async_copy(buf, acc.at[idx], sem, add=True)

        def gwait(buf, sem):
            pltpu.make_async_copy(gcat_ref.at[sbuf.at[pl.ds(0, CH)]],
                                  buf, sem).wait()

        def swait(buf, sem):
            pltpu.make_async_copy(buf, acc.at[dbuf.at[pl.ds(0, CH)]],
                                  sem).wait()

        gather(0, buf0, sem0)
        gather(1, buf1, sem1)

        def win2(i, _):
            w = i * 2
            gwait(buf0, sem0)
            scat_add(w, buf0, ssem0)
            gwait(buf1, sem1)
            scat_add(w + 1, buf1, ssem1)

            @pl.when(w + 2 < IWIN)
            def _():
                swait(buf0, ssem0)
                gather(w + 2, buf0, sem0)
                swait(buf1, ssem1)
                gather(w + 3, buf1, sem1)

            return 0

        lax.fori_loop(0, IWIN // 2, win2, 0)
        swait(buf0, ssem0)
        swait(buf1, ssem1)

    pltpu.sync_copy(src2_ref.at[pl.ds(ebase, ICH)], sbuf0)
    pltpu.sync_copy(dst2_ref.at[pl.ds(ebase, ICH)], dbuf0)

    def chunk2(k2, _):
        k = k2 * 2
        prefetch(k + 1, sbuf1, dbuf1)
        run_windows(sbuf0, dbuf0)
        iwait(sbuf1, dbuf1)

        @pl.when(k + 2 < NCHUNK)
        def _():
            prefetch(k + 2, sbuf0, dbuf0)

        run_windows(sbuf1, dbuf1)

        @pl.when(k + 2 < NCHUNK)
        def _():
            iwait(sbuf0, dbuf0)

        return 0

    lax.fori_loop(0, NCHUNK // 2, chunk2, 0)
    plsc.subcore_barrier()
    pltpu.sync_copy(acc.at[pl.ds(r0, ROWS_PER_TILE)],
                    scat_ref.at[pl.ds(grow0, ROWS_PER_TILE)])


def _t1a_body(x_ref, nu_ref, wga_ref, bga_ref, wgb_ref,
              bgb_ref, w1_ref, b1_ref, u_ref):
    nu = nu_ref[0, 0]
    t = jnp.maximum(nu * wga_ref[...] + bga_ref[...], 0.0)
    logits = jnp.dot(t, wgb_ref[...], preferred_element_type=jnp.float32)
    logits = logits + bgb_ref[...]
    m = jnp.max(logits, axis=-1, keepdims=True)
    ex = jnp.exp(logits - m)
    fw = ex / jnp.sum(ex, axis=-1, keepdims=True)
    h0 = x_ref[...] * fw
    u = jnp.dot(h0, w1_ref[...], preferred_element_type=jnp.float32)
    u_ref[...] = u + b1_ref[...]


def _t1b_body(u_ref, d0_ref, d1_ref, g_ref, dis_ref):
    deg = d0_ref[...] + d1_ref[...] + 1.0
    dis = lax.rsqrt(jnp.maximum(deg, 1.0))
    dis_ref[...] = dis
    g = u_ref[...] * dis
    g_ref[0] = g[:, :HH]
    g_ref[1] = g[:, HH:]


def _t23_body(s_ref, dis_ref, w_ref, b_ref, g_ref):
    dis = dis_ref[...]
    s_full = jnp.concatenate([s_ref[0], s_ref[1]], axis=1)
    h = jnp.maximum(s_full * dis, 0.0)
    u = jnp.dot(h, w_ref[...], preferred_element_type=jnp.float32)
    u = u + b_ref[...]
    g = u * dis
    g_ref[0] = g[:, :HH]
    g_ref[1] = g[:, HH:]


def _t4_body(s_ref, dis_ref, nu_ref, wa1h_ref, wa1nu_ref, ba1_ref, wa2_ref,
             ba2_ref, wo1_ref, bo1_ref, wo2_ref, bo2_ref, out_ref):
    dis = dis_ref[...]
    h3 = jnp.concatenate([s_ref[0], s_ref[1]], axis=1) * dis
    nu = nu_ref[0, 0]
    a = jnp.dot(h3, wa1h_ref[...], preferred_element_type=jnp.float32)
    a = jnp.maximum(a + nu * wa1nu_ref[...] + ba1_ref[...], 0.0)
    att_l = jnp.dot(a, wa2_ref[...], preferred_element_type=jnp.float32)
    att = jax.nn.sigmoid(att_l + ba2_ref[...])
    ah = h3 * att
    z = jnp.dot(ah, wo1_ref[...], preferred_element_type=jnp.float32)
    z = jnp.maximum(z + bo1_ref[...], 0.0)
    o = jnp.dot(z, wo2_ref[...], preferred_element_type=jnp.float32)
    out_ref[...] = jax.nn.sigmoid(o + bo2_ref[...])


def _full(shape):
    return pl.BlockSpec(shape, lambda i: tuple(0 for _ in shape))


_t1a = pl.pallas_call(
    _t1a_body,
    grid=(GB,),
    in_specs=[
        pl.BlockSpec((R, D), lambda i: (i, 0)),
        _full((1, 1)),
        _full((1, 32)),
        _full((1, 32)),
        _full((32, D)),
        _full((1, D)),
        _full((D, H)),
        _full((1, H)),
    ],
    out_specs=pl.BlockSpec((R, H), lambda i: (i, 0)),
    out_shape=jax.ShapeDtypeStruct((NP, H), jnp.float32),
)

_t1b = pl.pallas_call(
    _t1b_body,
    grid=(GB,),
    in_specs=[
        pl.BlockSpec((R, H), lambda i: (i, 0)),
        pl.BlockSpec((R, 1), lambda i: (i, 0)),
        pl.BlockSpec((R, 1), lambda i: (i, 0)),
    ],
    out_specs=[
        pl.BlockSpec((2, R, HH), lambda i: (0, i, 0)),
        pl.BlockSpec((R, 1), lambda i: (i, 0)),
    ],
    out_shape=[
        jax.ShapeDtypeStruct((2, NP, HH), jnp.float32),
        jax.ShapeDtypeStruct((NP, 1), jnp.float32),
    ],
)

_t23 = pl.pallas_call(
    _t23_body,
    grid=(GB,),
    in_specs=[
        pl.BlockSpec((2, R, HH), lambda i: (0, i, 0)),
        pl.BlockSpec((R, 1), lambda i: (i, 0)),
        _full((H, H)),
        _full((1, H)),
    ],
    out_specs=pl.BlockSpec((2, R, HH), lambda i: (0, i, 0)),
    out_shape=jax.ShapeDtypeStruct((2, NP, HH), jnp.float32),
)

_t4 = pl.pallas_call(
    _t4_body,
    grid=(GB,),
    in_specs=[
        pl.BlockSpec((2, R, HH), lambda i: (0, i, 0)),
        pl.BlockSpec((R, 1), lambda i: (i, 0)),
        _full((1, 1)),
        _full((H, HH)),
        _full((1, HH)),
        _full((1, HH)),
        _full((HH, 1)),
        _full((1, 1)),
        _full((H, HH)),
        _full((1, HH)),
        _full((HH, 1)),
        _full((1, 1)),
    ],
    out_specs=pl.BlockSpec((R, 1), lambda i: (i, 0)),
    out_shape=jax.ShapeDtypeStruct((NP, 1), jnp.float32),
)


def _eb_body(ei_ref, ps_ref, pd_ref, src2_ref, dst2_ref):
    i = pl.program_id(0)
    off = jnp.where(i >= 4, NP, 0).astype(jnp.int32)
    ept0 = E // NTILE
    for t in range(4):
        src2_ref[pl.ds(t * EPT, ept0)] = ei_ref[0, pl.ds(t * ept0, ept0)] + off
        dst2_ref[pl.ds(t * EPT, ept0)] = ei_ref[1, pl.ds(t * ept0, ept0)]
        src2_ref[pl.ds(t * EPT + ept0, CH_PAD)] = ps_ref[...] + off
        dst2_ref[pl.ds(t * EPT + ept0, CH_PAD)] = pd_ref[...]


_edge_build = pl.pallas_call(
    _eb_body,
    grid=(8,),
    in_specs=[
        pl.BlockSpec((2, 4 * (E // NTILE)), lambda i: (0, i % 4)),
        pl.BlockSpec((CH_PAD,), lambda i: (0,)),
        pl.BlockSpec((CH_PAD,), lambda i: (0,)),
    ],
    out_specs=[
        pl.BlockSpec((4 * EPT,), lambda i: (i,)),
        pl.BlockSpec((4 * EPT,), lambda i: (i,)),
    ],
    out_shape=[
        jax.ShapeDtypeStruct((2 * NTILE * EPT,), jnp.int32),
        jax.ShapeDtypeStruct((2 * NTILE * EPT,), jnp.int32),
    ],
)


def kernel(x, edge_index, nu, Wg_a, bg_a, Wg_b, bg_b, W1, b1, W2, b2, W3, b3,
           Wa1, ba1, Wa2, ba2, Wo1, bo1, Wo2, bo2):
    ei = edge_index.astype(jnp.int32)
    pad_s = jnp.arange(CH_PAD, dtype=jnp.int32) % N
    pad_d = N + jnp.arange(CH_PAD, dtype=jnp.int32) % (NP - N)
    src2, dst2 = _edge_build(ei, pad_s, pad_d)
    xp = jnp.pad(x, ((0, NP - N), (0, 0)))

    part = _deg_sc(dst2)
    d0 = part[0:NP].reshape(NP, 1)
    d1 = part[NP:2 * NP].reshape(NP, 1)

    nu2 = nu.reshape(1, 1)
    u1 = _t1a(xp, nu2, Wg_a, bg_a.reshape(1, 32), Wg_b,
              bg_b.reshape(1, D), W1, b1.reshape(1, H))
    g, dis = _t1b(u1, d0, d1)

    s = _conv_sc(g.reshape(2 * NP, HH), src2, dst2).reshape(2, NP, HH)
    g = _t23(s, dis, W2, b2.reshape(1, H))
    s = _conv_sc(g.reshape(2 * NP, HH), src2, dst2).reshape(2, NP, HH)
    g = _t23(s, dis, W3, b3.reshape(1, H))
    s = _conv_sc(g.reshape(2 * NP, HH), src2, dst2).reshape(2, NP, HH)

    out = _t4(s, dis, nu2, Wa1[:H], Wa1[H:H + 1], ba1.reshape(1, HH),
              Wa2, ba2.reshape(1, 1), Wo1, bo1.reshape(1, HH),
              Wo2, bo2.reshape(1, 1))
    return out[:N]


# speedup vs baseline: 15.9124x; 1.0028x over previous
import functools

import jax
import jax.numpy as jnp
from jax import lax
from jax.experimental import pallas as pl
from jax.experimental.pallas import tpu as pltpu
from jax.experimental.pallas import tpu_sc as plsc

N = 10000
NP = 10240
E = 320000
D = 128
H = 256
HH = H // 2

NTILE = 16
ROWS_PER_TILE = NP // NTILE
CH = 128
WPT = 160
EPT = CH * WPT
CH_PAD = EPT - E // NTILE
ICH = 2560
IWIN = ICH // CH
NCHUNK = EPT // ICH

DEG_PER_TILE = NP // NTILE
DEG_EDGES_PER_TILE = EPT // 2
DEG_NWIN = DEG_EDGES_PER_TILE // CH

R = 1280
GB = NP // R

_sc_mesh = plsc.VectorSubcoreMesh(core_axis_name="c", subcore_axis_name="s")


@functools.partial(
    pl.kernel,
    out_type=jax.ShapeDtypeStruct((2 * NP,), jnp.float32),
    mesh=_sc_mesh,
    scratch_types=[
        pltpu.VMEM_SHARED((NP,), jnp.float32),
        pltpu.VMEM((DEG_EDGES_PER_TILE,), jnp.int32),
        pltpu.VMEM((CH,), jnp.float32),
        pltpu.VMEM((DEG_PER_TILE,), jnp.float32),
        pltpu.SemaphoreType.DMA,
    ],
)
def _deg_sc(dst_ref, part_ref, acc, idxb, ones_b, zbuf, ssem):
    c = lax.axis_index("c")
    sid = lax.axis_index("s")

    def fill_z(i, _):
        zbuf[pl.ds(i * 16, 16)] = jnp.zeros((16,), jnp.float32)
        return 0

    lax.fori_loop(0, DEG_PER_TILE // 16, fill_z, 0)

    def fill_o(i, _):
        ones_b[pl.ds(i * 16, 16)] = jnp.ones((16,), jnp.float32)
        return 0

    lax.fori_loop(0, CH // 16, fill_o, 0)

    base = pl.multiple_of(sid * EPT + c * DEG_EDGES_PER_TILE, 8)
    pltpu.sync_copy(dst_ref.at[pl.ds(base, DEG_EDGES_PER_TILE)], idxb)

    my0 = pl.multiple_of(sid * DEG_PER_TILE, 8)
    pltpu.sync_copy(zbuf, acc.at[pl.ds(my0, DEG_PER_TILE)])
    plsc.subcore_barrier()

    def win(w, _):
        idx = idxb.at[pl.ds(pl.multiple_of(w * CH, 8), CH)]
        pltpu.async_copy(ones_b, acc.at[idx], ssem, add=True)
        return 0

    lax.fori_loop(0, DEG_NWIN, win, 0)

    def drain(w, _):
        idx = idxb.at[pl.ds(0, CH)]
        pltpu.make_async_copy(ones_b, acc.at[idx], ssem).wait()
        return 0

    lax.fori_loop(0, DEG_NWIN, drain, 0)
    plsc.subcore_barrier()
    out0 = pl.multiple_of(c * NP + sid * DEG_PER_TILE, 8)
    pltpu.sync_copy(acc.at[pl.ds(my0, DEG_PER_TILE)],
                    part_ref.at[pl.ds(out0, DEG_PER_TILE)])


@functools.partial(
    pl.kernel,
    out_type=jax.ShapeDtypeStruct((2 * NP, HH), jnp.float32),
    mesh=_sc_mesh,
    scratch_types=[
        pltpu.VMEM_SHARED((NP, HH), jnp.float32),
        pltpu.VMEM((ICH,), jnp.int32),
        pltpu.VMEM((ICH,), jnp.int32),
        pltpu.VMEM((ICH,), jnp.int32),
        pltpu.VMEM((ICH,), jnp.int32),
        pltpu.VMEM((CH, HH), jnp.float32),
        pltpu.VMEM((CH, HH), jnp.float32),
        pltpu.SemaphoreType.DMA,
        pltpu.SemaphoreType.DMA,
        pltpu.SemaphoreType.DMA,
        pltpu.SemaphoreType.DMA,
        pltpu.SemaphoreType.DMA,
    ],
)
def _conv_sc(gcat_ref, src2_ref, dst2_ref, scat_ref, acc, sbuf0, dbuf0,
             sbuf1, dbuf1, buf0, buf1, sem0, sem1, ssem0, ssem1, isem):
    c = lax.axis_index("c")
    sid = lax.axis_index("s")

    r0 = pl.multiple_of(sid * ROWS_PER_TILE, 8)
    grow0 = pl.multiple_of(c * NP + sid * ROWS_PER_TILE, 8)
    ebase = pl.multiple_of((c * NTILE + sid) * EPT, 8)

    pltpu.sync_copy(gcat_ref.at[pl.ds(grow0, ROWS_PER_TILE)],
                    acc.at[pl.ds(r0, ROWS_PER_TILE)])
    plsc.subcore_barrier()

    def prefetch(k, sb, db):
        eoff = pl.multiple_of(ebase + k * ICH, 8)
        pltpu.async_copy(src2_ref.at[pl.ds(eoff, ICH)], sb, isem)
        pltpu.async_copy(dst2_ref.at[pl.ds(eoff, ICH)], db, isem)

    def iwait(sb, db):
        pltpu.make_async_copy(src2_ref.at[pl.ds(ebase, ICH)], sb, isem).wait()
        pltpu.make_async_copy(dst2_ref.at[pl.ds(ebase, ICH)], db, isem).wait()

    def run_windows(sbuf, dbuf):
        def gather(w, buf, sem):
            idx = sbuf.at[pl.ds(pl.multiple_of(w * CH, 8), CH)]
            pltpu.async_copy(gcat_ref.at[idx], buf, sem)

        def scat_add(w, buf, sem):
            idx = dbuf.at[pl.ds(pl.multiple_of(w * CH, 8), CH)]
            pltpu.async_copy(buf, acc.at[idx], sem, add=True)

        def gwait(buf, sem):
            pltpu.make_async_copy(gcat_ref.at[sbuf.at[pl.ds(0, CH)]],
                                  buf, sem).wait()

        def swait(buf, sem):
            pltpu.make_async_copy(buf, acc.at[dbuf.at[pl.ds(0, CH)]],
                                  sem).wait()

        gather(0, buf0, sem0)
        gather(1, buf1, sem1)

        def win2(i, _):
            w = i * 2
            gwait(buf0, sem0)
            scat_add(w, buf0, ssem0)
            gwait(buf1, sem1)
            scat_add(w + 1, buf1, ssem1)

            @pl.when(w + 2 < IWIN)
            def _():
                swait(buf0, ssem0)
                gather(w + 2, buf0, sem0)
                swait(buf1, ssem1)
                gather(w + 3, buf1, sem1)

            return 0

        lax.fori_loop(0, IWIN // 2, win2, 0)
        swait(buf0, ssem0)
        swait(buf1, ssem1)

    pltpu.sync_copy(src2_ref.at[pl.ds(ebase, ICH)], sbuf0)
    pltpu.sync_copy(dst2_ref.at[pl.ds(ebase, ICH)], dbuf0)

    def chunk2(k2, _):
        k = k2 * 2
        prefetch(k + 1, sbuf1, dbuf1)
        run_windows(sbuf0, dbuf0)
        iwait(sbuf1, dbuf1)

        @pl.when(k + 2 < NCHUNK)
        def _():
            prefetch(k + 2, sbuf0, dbuf0)

        run_windows(sbuf1, dbuf1)

        @pl.when(k + 2 < NCHUNK)
        def _():
            iwait(sbuf0, dbuf0)

        return 0

    lax.fori_loop(0, NCHUNK // 2, chunk2, 0)
    plsc.subcore_barrier()
    pltpu.sync_copy(acc.at[pl.ds(r0, ROWS_PER_TILE)],
                    scat_ref.at[pl.ds(grow0, ROWS_PER_TILE)])


def _t1a_body(x_ref, nu_ref, wga_ref, bga_ref, wgb_ref,
              bgb_ref, w1_ref, b1_ref, u_ref):
    nu = nu_ref[0, 0]
    t = jnp.maximum(nu * wga_ref[...] + bga_ref[...], 0.0)
    logits = jnp.dot(t, wgb_ref[...], preferred_element_type=jnp.float32)
    logits = logits + bgb_ref[...]
    m = jnp.max(logits, axis=-1, keepdims=True)
    ex = jnp.exp(logits - m)
    fw = ex / jnp.sum(ex, axis=-1, keepdims=True)
    h0 = x_ref[...] * fw
    u = jnp.dot(h0, w1_ref[...], preferred_element_type=jnp.float32)
    u_ref[...] = u + b1_ref[...]


def _t1b_body(u_ref, d0_ref, d1_ref, g_ref, dis_ref):
    deg = d0_ref[...] + d1_ref[...] + 1.0
    dis = lax.rsqrt(jnp.maximum(deg, 1.0))
    dis_ref[...] = dis
    g = u_ref[...] * dis
    g_ref[0] = g[:, :HH]
    g_ref[1] = g[:, HH:]


def _t23_body(s_ref, dis_ref, w_ref, b_ref, g_ref):
    dis = dis_ref[...]
    s_full = jnp.concatenate([s_ref[0], s_ref[1]], axis=1)
    h = jnp.maximum(s_full * dis, 0.0)
    u = jnp.dot(h, w_ref[...], preferred_element_type=jnp.float32)
    u = u + b_ref[...]
    g = u * dis
    g_ref[0] = g[:, :HH]
    g_ref[1] = g[:, HH:]


def _t4_body(s_ref, dis_ref, nu_ref, wa1h_ref, wa1nu_ref, ba1_ref, wa2_ref,
             ba2_ref, wo1_ref, bo1_ref, wo2_ref, bo2_ref, out_ref):
    dis = dis_ref[...]
    h3 = jnp.concatenate([s_ref[0], s_ref[1]], axis=1) * dis
    nu = nu_ref[0, 0]
    a = jnp.dot(h3, wa1h_ref[...], preferred_element_type=jnp.float32)
    a = jnp.maximum(a + nu * wa1nu_ref[...] + ba1_ref[...], 0.0)
    att_l = jnp.dot(a, wa2_ref[...], preferred_element_type=jnp.float32)
    att = jax.nn.sigmoid(att_l + ba2_ref[...])
    ah = h3 * att
    z = jnp.dot(ah, wo1_ref[...], preferred_element_type=jnp.float32)
    z = jnp.maximum(z + bo1_ref[...], 0.0)
    o = jnp.dot(z, wo2_ref[...], preferred_element_type=jnp.float32)
    out_ref[...] = jax.nn.sigmoid(o + bo2_ref[...])


def _full(shape):
    return pl.BlockSpec(shape, lambda i: tuple(0 for _ in shape))


_t1a = pl.pallas_call(
    _t1a_body,
    grid=(GB,),
    in_specs=[
        pl.BlockSpec((R, D), lambda i: (i, 0)),
        _full((1, 1)),
        _full((1, 32)),
        _full((1, 32)),
        _full((32, D)),
        _full((1, D)),
        _full((D, H)),
        _full((1, H)),
    ],
    out_specs=pl.BlockSpec((R, H), lambda i: (i, 0)),
    out_shape=jax.ShapeDtypeStruct((NP, H), jnp.float32),
)

_t1b = pl.pallas_call(
    _t1b_body,
    grid=(GB,),
    in_specs=[
        pl.BlockSpec((R, H), lambda i: (i, 0)),
        pl.BlockSpec((R, 1), lambda i: (i, 0)),
        pl.BlockSpec((R, 1), lambda i: (i, 0)),
    ],
    out_specs=[
        pl.BlockSpec((2, R, HH), lambda i: (0, i, 0)),
        pl.BlockSpec((R, 1), lambda i: (i, 0)),
    ],
    out_shape=[
        jax.ShapeDtypeStruct((2, NP, HH), jnp.float32),
        jax.ShapeDtypeStruct((NP, 1), jnp.float32),
    ],
)

_t23 = pl.pallas_call(
    _t23_body,
    grid=(GB,),
    in_specs=[
        pl.BlockSpec((2, R, HH), lambda i: (0, i, 0)),
        pl.BlockSpec((R, 1), lambda i: (i, 0)),
        _full((H, H)),
        _full((1, H)),
    ],
    out_specs=pl.BlockSpec((2, R, HH), lambda i: (0, i, 0)),
    out_shape=jax.ShapeDtypeStruct((2, NP, HH), jnp.float32),
)

_t4 = pl.pallas_call(
    _t4_body,
    grid=(GB,),
    in_specs=[
        pl.BlockSpec((2, R, HH), lambda i: (0, i, 0)),
        pl.BlockSpec((R, 1), lambda i: (i, 0)),
        _full((1, 1)),
        _full((H, HH)),
        _full((1, HH)),
        _full((1, HH)),
        _full((HH, 1)),
        _full((1, 1)),
        _full((H, HH)),
        _full((1, HH)),
        _full((HH, 1)),
        _full((1, 1)),
    ],
    out_specs=pl.BlockSpec((R, 1), lambda i: (i, 0)),
    out_shape=jax.ShapeDtypeStruct((N, 1), jnp.float32),
)


def _eb_body(ei_ref, ps_ref, pd_ref, src2_ref, dst2_ref):
    i = pl.program_id(0)
    off = jnp.where(i >= 4, NP, 0).astype(jnp.int32)
    ept0 = E // NTILE
    for t in range(4):
        src2_ref[pl.ds(t * EPT, ept0)] = ei_ref[0, pl.ds(t * ept0, ept0)] + off
        dst2_ref[pl.ds(t * EPT, ept0)] = ei_ref[1, pl.ds(t * ept0, ept0)]
        src2_ref[pl.ds(t * EPT + ept0, CH_PAD)] = ps_ref[...] + off
        dst2_ref[pl.ds(t * EPT + ept0, CH_PAD)] = pd_ref[...]


_edge_build = pl.pallas_call(
    _eb_body,
    grid=(8,),
    in_specs=[
        pl.BlockSpec((2, 4 * (E // NTILE)), lambda i: (0, i % 4)),
        pl.BlockSpec((CH_PAD,), lambda i: (0,)),
        pl.BlockSpec((CH_PAD,), lambda i: (0,)),
    ],
    out_specs=[
        pl.BlockSpec((4 * EPT,), lambda i: (i,)),
        pl.BlockSpec((4 * EPT,), lambda i: (i,)),
    ],
    out_shape=[
        jax.ShapeDtypeStruct((2 * NTILE * EPT,), jnp.int32),
        jax.ShapeDtypeStruct((2 * NTILE * EPT,), jnp.int32),
    ],
)


def kernel(x, edge_index, nu, Wg_a, bg_a, Wg_b, bg_b, W1, b1, W2, b2, W3, b3,
           Wa1, ba1, Wa2, ba2, Wo1, bo1, Wo2, bo2):
    ei = edge_index.astype(jnp.int32)
    pad_s = jnp.arange(CH_PAD, dtype=jnp.int32) % N
    pad_d = N + jnp.arange(CH_PAD, dtype=jnp.int32) % (NP - N)
    src2, dst2 = _edge_build(ei, pad_s, pad_d)
    xp = jnp.pad(x, ((0, NP - N), (0, 0)))

    part = _deg_sc(dst2)
    d0 = part[0:NP].reshape(NP, 1)
    d1 = part[NP:2 * NP].reshape(NP, 1)

    nu2 = nu.reshape(1, 1)
    u1 = _t1a(xp, nu2, Wg_a, bg_a.reshape(1, 32), Wg_b,
              bg_b.reshape(1, D), W1, b1.reshape(1, H))
    g, dis = _t1b(u1, d0, d1)

    s = _conv_sc(g.reshape(2 * NP, HH), src2, dst2).reshape(2, NP, HH)
    g = _t23(s, dis, W2, b2.reshape(1, H))
    s = _conv_sc(g.reshape(2 * NP, HH), src2, dst2).reshape(2, NP, HH)
    g = _t23(s, dis, W3, b3.reshape(1, H))
    s = _conv_sc(g.reshape(2 * NP, HH), src2, dst2).reshape(2, NP, HH)

    out = _t4(s, dis, nu2, Wa1[:H], Wa1[H:H + 1], ba1.reshape(1, HH),
              Wa2, ba2.reshape(1, 1), Wo1, bo1.reshape(1, HH),
              Wo2, bo2.reshape(1, 1))
    return out
